# baseline (speedup 1.0000x reference)
import functools

import jax
import jax.numpy as jnp
from jax import lax
from jax.experimental import pallas as pl
from jax.experimental.pallas import tpu as pltpu
from jax.experimental.pallas import tpu_sc as plsc

NN = 50000
EE = 800000
DIN = 128
DE = 16
HH = 64
NEG = 0.2
NC = 2
NS = 16
NWK = NC * NS
EPT = EE // NWK

NCH1 = 195
EB1 = NCH1 * 128
XBASE = NWK * EB1
NROW = EE // 128

_MESH = plsc.VectorSubcoreMesh(core_axis_name="c", subcore_axis_name="s")

_F32 = jnp.float32
_I32 = jnp.int32

_CP = 3128
_CP_LAST = NN - (NS - 1) * _CP
_ZR = 200


def _fill_vec(ref, rows, cols, vec):
    nslot = cols // 16

    def body(i, _):
        r = i // nslot
        s = i % nslot
        ref[r, pl.ds(s * 16, 16)] = vec
        return 0

    lax.fori_loop(0, rows * nslot, body, 0)


def _zero_idx(ref, n):
    z = jnp.zeros((16,), _I32)
    for i in range(n // 16):
        ref[pl.ds(i * 16, 16)] = z


def _zero_spmem(acc, zb, sid):
    nchunk = NN // _ZR

    def body(j, _):
        c = sid + NS * j

        @pl.when(c < nchunk)
        def _():
            r0 = pl.multiple_of(c * _ZR, 8)
            pltpu.sync_copy(zb, acc.at[pl.ds(r0, _ZR)])
        return 0

    lax.fori_loop(0, (nchunk + NS - 1) // NS, body, 0)


def _copy_out_rows(acc, out_at, sid):
    @pl.when(sid < NS - 1)
    def _():
        r0 = pl.multiple_of(sid * _CP, 8)
        pltpu.sync_copy(acc.at[pl.ds(r0, _CP)], out_at.at[pl.ds(r0, _CP)])

    @pl.when(sid == NS - 1)
    def _():
        r0 = (NS - 1) * _CP
        pltpu.sync_copy(acc.at[pl.ds(r0, _CP_LAST)],
                        out_at.at[pl.ds(r0, _CP_LAST)])


@functools.partial(
    pl.kernel,
    out_type=(
        jax.ShapeDtypeStruct((NC, NN, DE), _F32),
        jax.ShapeDtypeStruct((NC, NN, DE), _F32),
    ),
    mesh=_MESH,
    compiler_params=pltpu.CompilerParams(needs_layout_passes=False,
                                         use_tc_tiling_on_sc=False),
    scratch_types=[
        pltpu.VMEM((128,), _I32),
        pltpu.VMEM((48,), _I32),
        pltpu.VMEM((128, DE), _F32),
        pltpu.VMEM((48, DE), _F32),
        pltpu.VMEM((_ZR, DE), _F32),
        pltpu.VMEM_SHARED((NN, DE), _F32),
    ],
)
def _pass0(dst_hbm, ea_hbm, outL, outD, idx, idxt, ea, eat, zb, acc):
    cid = lax.axis_index("c")
    sid = lax.axis_index("s")
    wid = sid * NC + cid
    base0 = wid * EPT
    zv = jnp.zeros((16,), _F32)
    ov = jnp.ones((16,), _F32)

    _fill_vec(zb, _ZR, DE, zv)
    _zero_idx(idxt, 48)
    pltpu.sync_copy(dst_hbm.at[pl.ds(base0 + 195 * 128, 40)],
                    idxt.at[pl.ds(0, 40)])

    _zero_spmem(acc, zb, sid)
    plsc.subcore_barrier()

    def chunk1(k, _):
        b = base0 + k * 128
        pltpu.sync_copy(dst_hbm.at[pl.ds(b, 128)], idx)
        pltpu.sync_copy(ea_hbm.at[pl.ds(b, 128)], ea)
        pltpu.sync_copy(ea, acc.at[idx], add=True)
        return 0
    lax.fori_loop(0, 195, chunk1, 0)

    _fill_vec(eat, 48, DE, zv)
    pltpu.sync_copy(ea_hbm.at[pl.ds(base0 + 195 * 128, 40)],
                    eat.at[pl.ds(0, 40)])
    pltpu.sync_copy(eat, acc.at[idxt], add=True)

    plsc.subcore_barrier()
    _copy_out_rows(acc, outL.at[cid], sid)
    plsc.subcore_barrier()

    _zero_spmem(acc, zb, sid)
    plsc.subcore_barrier()

    _fill_vec(ea, 128, DE, ov)

    def chunk2(k, _):
        b = base0 + k * 128
        pltpu.sync_copy(dst_hbm.at[pl.ds(b, 128)], idx)
        pltpu.sync_copy(ea, acc.at[idx], add=True)
        return 0
    lax.fori_loop(0, 195, chunk2, 0)

    _fill_vec(eat, 40, DE, ov)
    pltpu.sync_copy(eat, acc.at[idxt], add=True)

    plsc.subcore_barrier()
    _copy_out_rows(acc, outD.at[cid], sid)


@functools.partial(
    pl.kernel,
    out_type=jax.ShapeDtypeStruct((EE, 80), _F32),
    mesh=_MESH,
    compiler_params=pltpu.CompilerParams(needs_layout_passes=False,
                                         use_tc_tiling_on_sc=False),
    scratch_types=[
        pltpu.VMEM((128,), _I32), pltpu.VMEM((128,), _I32),
        pltpu.VMEM((128,), _I32), pltpu.VMEM((128,), _I32),
        pltpu.VMEM((128,), _I32), pltpu.VMEM((128,), _I32),
        pltpu.VMEM((128,), _I32), pltpu.VMEM((128,), _I32),
        pltpu.VMEM((128, HH), _F32), pltpu.VMEM((128, HH), _F32),
        pltpu.VMEM((128, HH), _F32), pltpu.VMEM((128, HH), _F32),
        pltpu.VMEM((128, HH), _F32), pltpu.VMEM((128, HH), _F32),
        pltpu.VMEM((128, 80), _F32), pltpu.VMEM((128, 80), _F32),
        pltpu.VMEM((HH, 16), _F32),
        pltpu.SemaphoreType.DMA,
        pltpu.SemaphoreType.DMA,
        pltpu.SemaphoreType.DMA,
    ],
)
def _pass1(src2, dst2, xl_hbm, xr_hbm, e_hbm, att_hbm, msg_out,
           S0, S1, S2, S3, D0, D1, D2, D3,
           bL0, bL1, bR0, bR1, bE0, bE1, bM0, bM1,
           attb, semI, semG, semO):
    cid = lax.axis_index("c")
    sid = lax.axis_index("s")
    wid = sid * NC + cid
    row0 = wid * NCH1
    eb0 = wid * EB1
    S = [S0, S1, S2, S3]
    D = [D0, D1, D2, D3]
    bL = [bL0, bL1]
    bR = [bR0, bR1]
    bE = [bE0, bE1]
    bM = [bM0, bM1]

    pltpu.sync_copy(att_hbm, attb)
    iot = lax.iota(_I32, 16)
    rows_list = [iot + g * 16 for g in range(8)]

    def fire_idx(x, j):
        pltpu.async_copy(src2.at[row0 + x], S[j], semI)
        pltpu.async_copy(dst2.at[row0 + x], D[j], semI)

    def drain_idx():
        pltpu.make_async_copy(src2.at[0], S[0], semI).wait()
        pltpu.make_async_copy(dst2.at[0], D[0], semI).wait()

    def fire_gather(x, j, p):
        b = pl.multiple_of(eb0 + x * 128, 8)
        pltpu.async_copy(xl_hbm.at[S[j]], bL[p], semG)
        pltpu.async_copy(xr_hbm.at[D[j]], bR[p], semG)
        pltpu.async_copy(e_hbm.at[pl.ds(b, 128)], bE[p], semG)

    def drain_gather(p):
        pltpu.make_async_copy(xl_hbm.at[S[0]], bL[p], semG).wait()
        pltpu.make_async_copy(xr_hbm.at[D[0]], bR[p], semG).wait()
        pltpu.make_async_copy(e_hbm.at[pl.ds(0, 128)], bE[p], semG).wait()

    def fire_out(x, p):
        b = pl.multiple_of(eb0 + x * 128, 8)
        pltpu.async_copy(bM[p], msg_out.at[pl.ds(b, 128)], semO)

    def drain_out(p):
        pltpu.make_async_copy(bM[p], msg_out.at[pl.ds(0, 128)], semO).wait()

    def compute(p):
        bLp, bRp, bEp, bMp = bL[p], bR[p], bE[p], bM[p]

        def dbody(d, accs):
            dv = jnp.full((16,), d, _I32)
            ad = plsc.load_gather(attb, [dv, iot])
            new = []
            for g in range(8):
                rows = rows_list[g]
                xld = plsc.load_gather(bLp, [rows, dv])
                xrd = plsc.load_gather(bRp, [rows, dv])
                ed = plsc.load_gather(bEp, [rows, dv])
                z = xld + xrd + ed
                z = jnp.maximum(z, NEG * z)
                new.append(accs[g] + ad * z)
            return tuple(new)

        accs = lax.fori_loop(0, HH, dbody,
                             tuple(jnp.zeros((16,), _F32) for _ in range(8)))
        exs = [jnp.exp(a) for a in accs]

        def mbody(d, _):
            for q in range(4):
                dv = jnp.full((16,), d, _I32) + q * DE
                for g in range(8):
                    rows = rows_list[g]
                    xld = plsc.load_gather(bLp, [rows, dv])
                    plsc.store_scatter(bMp, [rows, dv], exs[g] * xld)
            return 0
        lax.fori_loop(0, DE, mbody, 0)

        def xbody(d, _):
            dv = jnp.full((16,), d, _I32) + 64
            for g in range(8):
                plsc.store_scatter(bMp, [rows_list[g], dv], exs[g])
            return 0
        lax.fori_loop(0, DE, xbody, 0)

    fire_idx(0, 0)
    fire_idx(1, 1)
    drain_idx()
    fire_gather(0, 0, 0)

    def body(k4, _):
        x0 = k4 * 4
        for j in range(4):
            x = x0 + j
            p = j % 2
            fire_idx(x + 2, (j + 2) % 4)
            drain_idx()
            fire_gather(x + 1, (j + 1) % 4, (p + 1) % 2)
            drain_gather(p)

            @pl.when(x >= 2)
            def _():
                drain_out(p)
            compute(p)
            fire_out(x, p)
        return 0
    lax.fori_loop(0, (NCH1 - 3) // 4, body, 0)

    fire_idx(194, 2)
    drain_idx()
    fire_gather(193, 1, 1)
    drain_gather(0)
    drain_out(0)
    compute(0)
    fire_out(192, 0)

    drain_idx()
    fire_gather(194, 2, 0)
    drain_gather(1)
    drain_out(1)
    compute(1)
    fire_out(193, 1)

    drain_gather(0)
    drain_out(0)
    compute(0)
    fire_out(194, 0)

    drain_out(1)
    drain_out(0)

    @pl.when(wid < 10)
    def _():
        pltpu.sync_copy(src2.at[NWK * NCH1 + wid], S[3])
        pltpu.sync_copy(dst2.at[NWK * NCH1 + wid], D[3])
        xb = pl.multiple_of(XBASE + wid * 128, 8)
        pltpu.async_copy(xl_hbm.at[S[3]], bL[1], semG)
        pltpu.async_copy(xr_hbm.at[D[3]], bR[1], semG)
        pltpu.async_copy(e_hbm.at[pl.ds(xb, 128)], bE[1], semG)
        drain_gather(1)
        compute(1)
        pltpu.sync_copy(bM[1], msg_out.at[pl.ds(xb, 128)])


@functools.partial(
    pl.kernel,
    out_type=jax.ShapeDtypeStruct((6, NN, DE), _F32),
    mesh=_MESH,
    compiler_params=pltpu.CompilerParams(needs_layout_passes=False,
                                         use_tc_tiling_on_sc=False),
    scratch_types=[
        pltpu.VMEM((128,), _I32), pltpu.VMEM((128,), _I32),
        pltpu.VMEM((128,), _I32), pltpu.VMEM((128,), _I32),
        pltpu.VMEM((128, DE), _F32), pltpu.VMEM((128, DE), _F32),
        pltpu.VMEM((128, DE), _F32), pltpu.VMEM((128, DE), _F32),
        pltpu.VMEM((_ZR, DE), _F32),
        pltpu.VMEM_SHARED((NN, DE), _F32),
        pltpu.SemaphoreType.DMA,
        pltpu.SemaphoreType.DMA,
        pltpu.SemaphoreType.DMA,
    ],
)
def _pass2(dst2, msg_hbm, acc_out,
           S0, S1, S2, S3, B0, B1, B2, B3, zb, accS, semI, semG, semS):
    cid = lax.axis_index("c")
    sid = lax.axis_index("s")
    S = [S0, S1, S2, S3]
    B = [B0, B1, B2, B3]
    _fill_vec(zb, _ZR, DE, jnp.zeros((16,), _F32))

    def run_phase(q, qout, row_base, eb_base, nch, n_extra, extra_row,
                  extra_eb):
        _zero_spmem(accS, zb, sid)
        plsc.subcore_barrier()

        def fire_idx(x, j):
            pltpu.async_copy(dst2.at[row_base + x], S[j], semI)

        def drain_idx():
            pltpu.make_async_copy(dst2.at[0], S[0], semI).wait()

        def fire_read(x, p):
            b = pl.multiple_of(eb_base + x * 128, 8)
            pltpu.async_copy(msg_hbm.at[pl.ds(b, 128), q], B[p], semG)

        def drain_read(p):
            pltpu.make_async_copy(msg_hbm.at[pl.ds(0, 128), q], B[p],
                                  semG).wait()

        def fire_scat(x, j, p):
            pltpu.async_copy(B[p], accS.at[S[j]], semS, add=True)

        def drain_scat():
            pltpu.make_async_copy(B[0], accS.at[S[0]], semS).wait()

        fire_idx(0, 0)
        fire_idx(1, 1)
        drain_idx()
        fire_read(0, 0)

        def body(k4, _):
            x0 = k4 * 4
            for j in range(4):
                x = x0 + j

                @pl.when(x >= 1)
                def _():
                    drain_scat()
                fire_idx(x + 2, (j + 2) % 4)
                drain_idx()
                fire_read(x + 1, (j + 1) % 4)
                drain_read(j)
                fire_scat(x, j, j)
            return 0
        nbody = (nch - 2) // 4
        lax.fori_loop(0, nbody, body, 0)

        for x in range(nbody * 4, nch):
            j = x % 4
            drain_scat()
            if x + 2 < nch:
                fire_idx(x + 2, (j + 2) % 4)
            if x + 1 < nch:
                drain_idx()
                fire_read(x + 1, (j + 1) % 4)
            drain_read(j)
            fire_scat(x, j, j)
        drain_scat()

        @pl.when(sid < n_extra)
        def _():
            pltpu.sync_copy(dst2.at[extra_row], S[0])
            xb = pl.multiple_of(extra_eb, 8)
            pltpu.sync_copy(msg_hbm.at[pl.ds(xb, 128), q], B[0])
            pltpu.sync_copy(B[0], accS.at[S[0]], add=True)

        plsc.subcore_barrier()
        _copy_out_rows(accS, acc_out.at[qout], sid)
        plsc.subcore_barrier()

    for ph in range(2):
        q = cid * 2 + ph
        run_phase(q, q,
                  sid * 390, sid * (390 * 128), 390,
                  10, NWK * NCH1 + sid, XBASE + sid * 128)

    run_phase(4, 4 + cid,
              cid * 3125 + sid * 195,
              cid * 400000 + sid * (195 * 128), 195,
              5, cid * 3125 + 3120 + sid,
              cid * 400000 + 399360 + sid * 128)


_RB = 1000
_REB = 2000


def _prep_body(x, wpt, bp, lng, lnb, wlt, bl, wrt, br, wet, attr, sl, sd,
               xl_o, xr_o, ex_o, lm_o):
    h = jnp.dot(x[...], wpt[...], preferred_element_type=_F32) + bp[...]
    mu = jnp.mean(h, axis=-1, keepdims=True)
    var = jnp.mean((h - mu) ** 2, axis=-1, keepdims=True)
    hn = (h - mu) * lax.rsqrt(var + 1e-5) * lng[...] + lnb[...]
    xl = jnp.dot(hn, wlt[...], preferred_element_type=_F32) + bl[...]
    xr = jnp.dot(hn, wrt[...], preferred_element_type=_F32) + br[...]
    deg = sd[...][0, :, 0:1] + sd[...][1, :, 0:1]
    lm = (sl[...][0] + sl[...][1]) / jnp.maximum(deg, 1.0)
    el = jnp.dot(lm, wet[...], preferred_element_type=_F32)
    z = xl + xr + el
    z = jnp.maximum(z, NEG * z)
    logit = jnp.sum(z * attr[...], axis=-1, keepdims=True)
    xl_o[...] = xl
    xr_o[...] = xr
    ex_o[...] = jnp.exp(logit)
    lm_o[...] = lm


def _finish_prep_body(acc, exs, xl, bias, lng, lnb, wlt, bl, wrt, br,
                      wet, attr, lm, xl_o, xr_o, ex_o):
    a = acc[...]
    num = jnp.concatenate([a[0], a[1], a[2], a[3]], axis=-1) + exs[...] * xl[...]
    d = a[4][:, 0:1] + a[5][:, 0:1] + exs[...] + 1e-16
    h = jnp.maximum(num / d + bias[...], 0.0)
    mu = jnp.mean(h, axis=-1, keepdims=True)
    var = jnp.mean((h - mu) ** 2, axis=-1, keepdims=True)
    hn = (h - mu) * lax.rsqrt(var + 1e-5) * lng[...] + lnb[...]
    xl2 = jnp.dot(hn, wlt[...], preferred_element_type=_F32) + bl[...]
    xr2 = jnp.dot(hn, wrt[...], preferred_element_type=_F32) + br[...]
    el = jnp.dot(lm[...], wet[...], preferred_element_type=_F32)
    z = xl2 + xr2 + el
    z = jnp.maximum(z, NEG * z)
    logit = jnp.sum(z * attr[...], axis=-1, keepdims=True)
    xl_o[...] = xl2
    xr_o[...] = xr2
    ex_o[...] = jnp.exp(logit)


def _finish_cls_body(acc, exs, xl, bias, wct, bc, cls_o, h_o):
    a = acc[...]
    num = jnp.concatenate([a[0], a[1], a[2], a[3]], axis=-1) + exs[...] * xl[...]
    d = a[4][:, 0:1] + a[5][:, 0:1] + exs[...] + 1e-16
    h = jnp.maximum(num / d + bias[...], 0.0)
    h_o[...] = h
    cls_o[...] = jnp.dot(h, wct[...], preferred_element_type=_F32) + bc[...]


def _edge_e_body(ea, w1, w2, e1_o, e2_o):
    a = ea[...]
    e1_o[...] = jnp.dot(a, w1[...], preferred_element_type=_F32)
    e2_o[...] = jnp.dot(a, w2[...], preferred_element_type=_F32)


def _row_spec(shape):
    nd = len(shape)
    if nd == 2:
        return pl.BlockSpec((_RB, shape[1]), lambda i: (i, 0))
    return pl.BlockSpec((shape[0], _RB, shape[2]), lambda i: (0, i, 0))


def _const_spec(shape):
    return pl.BlockSpec(shape, lambda i: tuple(0 for _ in shape))


def _tc_edge_e(ea, wet1, wet2):
    return pl.pallas_call(
        _edge_e_body,
        grid=(EE // _REB,),
        in_specs=[pl.BlockSpec((_REB, DE), lambda i: (i, 0)),
                  _const_spec((DE, HH)), _const_spec((DE, HH))],
        out_specs=[pl.BlockSpec((_REB, HH), lambda i: (i, 0)),
                   pl.BlockSpec((_REB, HH), lambda i: (i, 0))],
        out_shape=[jax.ShapeDtypeStruct((EE, HH), _F32),
                   jax.ShapeDtypeStruct((EE, HH), _F32)],
    )(ea, wet1, wet2)


def _tc_prep(x, wpt, bp, lng, lnb, wlt, bl, wrt, br, wet, attr, sumL, sumD):
    return pl.pallas_call(
        _prep_body,
        grid=(NN // _RB,),
        in_specs=[pl.BlockSpec((_RB, DIN), lambda i: (i, 0)),
                  _const_spec((DIN, HH)), _const_spec((1, HH)),
                  _const_spec((1, HH)), _const_spec((1, HH)),
                  _const_spec((HH, HH)), _const_spec((1, HH)),
                  _const_spec((HH, HH)), _const_spec((1, HH)),
                  _const_spec((DE, HH)), _const_spec((1, HH)),
                  _row_spec((NC, NN, DE)), _row_spec((NC, NN, DE))],
        out_specs=[pl.BlockSpec((_RB, HH), lambda i: (i, 0)),
                   pl.BlockSpec((_RB, HH), lambda i: (i, 0)),
                   pl.BlockSpec((_RB, 1), lambda i: (i, 0)),
                   pl.BlockSpec((_RB, DE), lambda i: (i, 0))],
        out_shape=[jax.ShapeDtypeStruct((NN, HH), _F32),
                   jax.ShapeDtypeStruct((NN, HH), _F32),
                   jax.ShapeDtypeStruct((NN, 1), _F32),
                   jax.ShapeDtypeStruct((NN, DE), _F32)],
    )(x, wpt, bp, lng, lnb, wlt, bl, wrt, br, wet, attr, sumL, sumD)


def _tc_finish_prep(acc, exs, xl, bias, lng, lnb, wlt, bl, wrt, br,
                    wet, attr, lm):
    return pl.pallas_call(
        _finish_prep_body,
        grid=(NN // _RB,),
        in_specs=[_row_spec((6, NN, DE)),
                  pl.BlockSpec((_RB, 1), lambda i: (i, 0)),
                  pl.BlockSpec((_RB, HH), lambda i: (i, 0)),
                  _const_spec((1, HH)),
                  _const_spec((1, HH)), _const_spec((1, HH)),
                  _const_spec((HH, HH)), _const_spec((1, HH)),
                  _const_spec((HH, HH)), _const_spec((1, HH)),
                  _const_spec((DE, HH)), _const_spec((1, HH)),
                  pl.BlockSpec((_RB, DE), lambda i: (i, 0))],
        out_specs=[pl.BlockSpec((_RB, HH), lambda i: (i, 0)),
                   pl.BlockSpec((_RB, HH), lambda i: (i, 0)),
                   pl.BlockSpec((_RB, 1), lambda i: (i, 0))],
        out_shape=[jax.ShapeDtypeStruct((NN, HH), _F32),
                   jax.ShapeDtypeStruct((NN, HH), _F32),
                   jax.ShapeDtypeStruct((NN, 1), _F32)],
    )(acc, exs, xl, bias, lng, lnb, wlt, bl, wrt, br, wet, attr, lm)


def _tc_finish_cls(acc, exs, xl, bias, wct, bc):
    return pl.pallas_call(
        _finish_cls_body,
        grid=(NN // _RB,),
        in_specs=[_row_spec((6, NN, DE)),
                  pl.BlockSpec((_RB, 1), lambda i: (i, 0)),
                  pl.BlockSpec((_RB, HH), lambda i: (i, 0)),
                  _const_spec((1, HH)),
                  _const_spec((HH, 64)), _const_spec((1, 64))],
        out_specs=[pl.BlockSpec((_RB, 64), lambda i: (i, 0)),
                   pl.BlockSpec((_RB, HH), lambda i: (i, 0))],
        out_shape=[jax.ShapeDtypeStruct((NN, 64), _F32),
                   jax.ShapeDtypeStruct((NN, HH), _F32)],
    )(acc, exs, xl, bias, wct, bc)


def kernel(x, edge_index, edge_attr, params):
    src = edge_index[0]
    dst = edge_index[1]
    src2 = src.reshape(NROW, 128)
    dst2 = dst.reshape(NROW, 128)
    p1, p2 = params['layers']

    wpt = params['Wp'].T
    bp = params['bp'].reshape(1, HH)
    wct = params['Wc'].T
    bc = params['bc'].reshape(1, 64)

    def layer_mats(p):
        return (p['ln_g'].reshape(1, HH), p['ln_b'].reshape(1, HH),
                p['Wl'].T, p['bl'].reshape(1, HH),
                p['Wr'].T, p['br'].reshape(1, HH),
                p['We'].T, p['att'].reshape(1, HH),
                jnp.broadcast_to(p['att'].reshape(HH, 1), (HH, 16)),
                p['bias'].reshape(1, HH))

    (lng1, lnb1, wlt1, bl1, wrt1, br1, wet1, attr1, attv1, bias1) = layer_mats(p1)
    (lng2, lnb2, wlt2, bl2, wrt2, br2, wet2, attr2, attv2, bias2) = layer_mats(p2)

    sumL, sumD = _pass0(dst, edge_attr)
    e1, e2 = _tc_edge_e(edge_attr, wet1, wet2)
    xl1, xr1, exs1, lm = _tc_prep(x, wpt, bp, lng1, lnb1, wlt1, bl1, wrt1,
                                  br1, wet1, attr1, sumL, sumD)
    msg1 = _pass1(src2, dst2, xl1, xr1, e1, attv1)
    acc1 = _pass2(dst2, msg1.reshape(EE, 5, DE))
    xl2, xr2, exs2 = _tc_finish_prep(acc1, exs1, xl1, bias1, lng2,
                                     lnb2, wlt2, bl2, wrt2, br2, wet2,
                                     attr2, lm)
    msg2 = _pass1(src2, dst2, xl2, xr2, e2, attv2)
    acc2 = _pass2(dst2, msg2.reshape(EE, 5, DE))
    cls, h = _tc_finish_cls(acc2, exs2, xl2, bias2, wct, bc)
    return (cls, h)


# speedup vs baseline: 1.4503x; 1.4503x over previous
import functools

import jax
import jax.numpy as jnp
from jax import lax
from jax.experimental import pallas as pl
from jax.experimental.pallas import tpu as pltpu
from jax.experimental.pallas import tpu_sc as plsc

NN = 50000
EE = 800000
DIN = 128
DE = 16
HH = 64
NEG = 0.2
NC = 2
NS = 16
NWK = NC * NS
EPT = EE // NWK

NCH1 = 195
EB1 = NCH1 * 128
XBASE = NWK * EB1
NROW = EE // 128

_MESH = plsc.VectorSubcoreMesh(core_axis_name="c", subcore_axis_name="s")

_F32 = jnp.float32
_I32 = jnp.int32

_CP = 3128
_CP_LAST = NN - (NS - 1) * _CP
_ZR = 200


def _fill_vec(ref, rows, cols, vec):
    nslot = cols // 16

    def body(i, _):
        r = i // nslot
        s = i % nslot
        ref[r, pl.ds(s * 16, 16)] = vec
        return 0

    lax.fori_loop(0, rows * nslot, body, 0)


def _zero_idx(ref, n):
    z = jnp.zeros((16,), _I32)
    for i in range(n // 16):
        ref[pl.ds(i * 16, 16)] = z


def _zero_spmem(acc, zb, sid):
    nchunk = NN // _ZR

    def body(j, _):
        c = sid + NS * j

        @pl.when(c < nchunk)
        def _():
            r0 = pl.multiple_of(c * _ZR, 8)
            pltpu.sync_copy(zb, acc.at[pl.ds(r0, _ZR)])
        return 0

    lax.fori_loop(0, (nchunk + NS - 1) // NS, body, 0)


def _copy_out_rows(acc, out_at, sid):
    @pl.when(sid < NS - 1)
    def _():
        r0 = pl.multiple_of(sid * _CP, 8)
        pltpu.sync_copy(acc.at[pl.ds(r0, _CP)], out_at.at[pl.ds(r0, _CP)])

    @pl.when(sid == NS - 1)
    def _():
        r0 = (NS - 1) * _CP
        pltpu.sync_copy(acc.at[pl.ds(r0, _CP_LAST)],
                        out_at.at[pl.ds(r0, _CP_LAST)])


@functools.partial(
    pl.kernel,
    out_type=(
        jax.ShapeDtypeStruct((NC, NN, DE), _F32),
        jax.ShapeDtypeStruct((NC, NN, DE), _F32),
    ),
    mesh=_MESH,
    compiler_params=pltpu.CompilerParams(needs_layout_passes=False,
                                         use_tc_tiling_on_sc=False),
    scratch_types=[
        pltpu.VMEM((128,), _I32),
        pltpu.VMEM((48,), _I32),
        pltpu.VMEM((128, DE), _F32),
        pltpu.VMEM((48, DE), _F32),
        pltpu.VMEM((_ZR, DE), _F32),
        pltpu.VMEM_SHARED((NN, DE), _F32),
    ],
)
def _pass0(dst_hbm, ea_hbm, outL, outD, idx, idxt, ea, eat, zb, acc):
    cid = lax.axis_index("c")
    sid = lax.axis_index("s")
    wid = sid * NC + cid
    base0 = wid * EPT
    zv = jnp.zeros((16,), _F32)
    ov = jnp.ones((16,), _F32)

    _fill_vec(zb, _ZR, DE, zv)
    _zero_idx(idxt, 48)
    pltpu.sync_copy(dst_hbm.at[pl.ds(base0 + 195 * 128, 40)],
                    idxt.at[pl.ds(0, 40)])

    _zero_spmem(acc, zb, sid)
    plsc.subcore_barrier()

    def chunk1(k, _):
        b = base0 + k * 128
        pltpu.sync_copy(dst_hbm.at[pl.ds(b, 128)], idx)
        pltpu.sync_copy(ea_hbm.at[pl.ds(b, 128)], ea)
        pltpu.sync_copy(ea, acc.at[idx], add=True)
        return 0
    lax.fori_loop(0, 195, chunk1, 0)

    _fill_vec(eat, 48, DE, zv)
    pltpu.sync_copy(ea_hbm.at[pl.ds(base0 + 195 * 128, 40)],
                    eat.at[pl.ds(0, 40)])
    pltpu.sync_copy(eat, acc.at[idxt], add=True)

    plsc.subcore_barrier()
    _copy_out_rows(acc, outL.at[cid], sid)
    plsc.subcore_barrier()

    _zero_spmem(acc, zb, sid)
    plsc.subcore_barrier()

    _fill_vec(ea, 128, DE, ov)

    def chunk2(k, _):
        b = base0 + k * 128
        pltpu.sync_copy(dst_hbm.at[pl.ds(b, 128)], idx)
        pltpu.sync_copy(ea, acc.at[idx], add=True)
        return 0
    lax.fori_loop(0, 195, chunk2, 0)

    _fill_vec(eat, 40, DE, ov)
    pltpu.sync_copy(eat, acc.at[idxt], add=True)

    plsc.subcore_barrier()
    _copy_out_rows(acc, outD.at[cid], sid)


@functools.partial(
    pl.kernel,
    out_type=jax.ShapeDtypeStruct((EE, 5, DE), _F32),
    mesh=_MESH,
    compiler_params=pltpu.CompilerParams(needs_layout_passes=False,
                                         use_tc_tiling_on_sc=False),
    scratch_types=[
        pltpu.VMEM((128,), _I32), pltpu.VMEM((128,), _I32),
        pltpu.VMEM((128,), _I32), pltpu.VMEM((128,), _I32),
        pltpu.VMEM((128,), _I32), pltpu.VMEM((128,), _I32),
        pltpu.VMEM((128,), _I32), pltpu.VMEM((128,), _I32),
        pltpu.VMEM((128, HH), _F32), pltpu.VMEM((128, HH), _F32),
        pltpu.VMEM((128, HH), _F32), pltpu.VMEM((128, HH), _F32),
        pltpu.VMEM((128, HH), _F32), pltpu.VMEM((128, HH), _F32),
        pltpu.VMEM((128, 5, DE), _F32), pltpu.VMEM((128, 5, DE), _F32),
        pltpu.VMEM((HH, 16), _F32),
        pltpu.SemaphoreType.DMA,
        pltpu.SemaphoreType.DMA,
        pltpu.SemaphoreType.DMA,
    ],
)
def _pass1(src2, dst2, xl_hbm, xr_hbm, e_hbm, att_hbm, msg_out,
           S0, S1, S2, S3, D0, D1, D2, D3,
           bL0, bL1, bR0, bR1, bE0, bE1, bM0, bM1,
           attb, semI, semG, semO):
    cid = lax.axis_index("c")
    sid = lax.axis_index("s")
    wid = sid * NC + cid
    row0 = wid * NCH1
    eb0 = wid * EB1
    S = [S0, S1, S2, S3]
    D = [D0, D1, D2, D3]
    bL = [bL0, bL1]
    bR = [bR0, bR1]
    bE = [bE0, bE1]
    bM = [bM0, bM1]

    pltpu.sync_copy(att_hbm, attb)
    iot = lax.iota(_I32, 16)
    rows_list = [iot + g * 16 for g in range(8)]

    def fire_idx(x, j):
        pltpu.async_copy(src2.at[row0 + x], S[j], semI)
        pltpu.async_copy(dst2.at[row0 + x], D[j], semI)

    def drain_idx():
        pltpu.make_async_copy(src2.at[0], S[0], semI).wait()
        pltpu.make_async_copy(dst2.at[0], D[0], semI).wait()

    def fire_gather(x, j, p):
        b = pl.multiple_of(eb0 + x * 128, 8)
        pltpu.async_copy(xl_hbm.at[S[j]], bL[p], semG)
        pltpu.async_copy(xr_hbm.at[D[j]], bR[p], semG)
        pltpu.async_copy(e_hbm.at[pl.ds(b, 128)], bE[p], semG)

    def drain_gather(p):
        pltpu.make_async_copy(xl_hbm.at[S[0]], bL[p], semG).wait()
        pltpu.make_async_copy(xr_hbm.at[D[0]], bR[p], semG).wait()
        pltpu.make_async_copy(e_hbm.at[pl.ds(0, 128)], bE[p], semG).wait()

    def fire_out(x, p):
        b = pl.multiple_of(eb0 + x * 128, 8)
        pltpu.async_copy(bM[p], msg_out.at[pl.ds(b, 128)], semO)

    def drain_out(p):
        pltpu.make_async_copy(bM[p], msg_out.at[pl.ds(0, 128)], semO).wait()

    def compute(p):
        bLp, bRp, bEp, bMp = bL[p], bR[p], bE[p], bM[p]

        def dbody(d, accs):
            dv = jnp.full((16,), d, _I32)
            ad = plsc.load_gather(attb, [dv, iot])
            new = []
            for g in range(8):
                rows = rows_list[g]
                xld = plsc.load_gather(bLp, [rows, dv])
                xrd = plsc.load_gather(bRp, [rows, dv])
                ed = plsc.load_gather(bEp, [rows, dv])
                z = xld + xrd + ed
                z = jnp.maximum(z, NEG * z)
                new.append(accs[g] + ad * z)
            return tuple(new)

        accs = lax.fori_loop(0, HH, dbody,
                             tuple(jnp.zeros((16,), _F32) for _ in range(8)))
        exs = [jnp.exp(a) for a in accs]

        def mbody(d, _):
            dv = jnp.full((16,), d, _I32)
            for q in range(4):
                qv = jnp.full((16,), q, _I32)
                cv = dv + q * DE
                for g in range(8):
                    rows = rows_list[g]
                    xld = plsc.load_gather(bLp, [rows, cv])
                    plsc.store_scatter(bMp, [rows, qv, dv], exs[g] * xld)
            return 0
        lax.fori_loop(0, DE, mbody, 0)

        qv4 = jnp.full((16,), 4, _I32)

        def xbody(d, _):
            dv = jnp.full((16,), d, _I32)
            for g in range(8):
                plsc.store_scatter(bMp, [rows_list[g], qv4, dv], exs[g])
            return 0
        lax.fori_loop(0, DE, xbody, 0)

    fire_idx(0, 0)
    fire_idx(1, 1)
    drain_idx()
    fire_gather(0, 0, 0)

    def body(k4, _):
        x0 = k4 * 4
        for j in range(4):
            x = x0 + j
            p = j % 2
            fire_idx(x + 2, (j + 2) % 4)
            drain_idx()
            fire_gather(x + 1, (j + 1) % 4, (p + 1) % 2)
            drain_gather(p)

            @pl.when(x >= 2)
            def _():
                drain_out(p)
            compute(p)
            fire_out(x, p)
        return 0
    lax.fori_loop(0, (NCH1 - 3) // 4, body, 0)

    fire_idx(194, 2)
    drain_idx()
    fire_gather(193, 1, 1)
    drain_gather(0)
    drain_out(0)
    compute(0)
    fire_out(192, 0)

    drain_idx()
    fire_gather(194, 2, 0)
    drain_gather(1)
    drain_out(1)
    compute(1)
    fire_out(193, 1)

    drain_gather(0)
    drain_out(0)
    compute(0)
    fire_out(194, 0)

    drain_out(1)
    drain_out(0)

    @pl.when(wid < 10)
    def _():
        pltpu.sync_copy(src2.at[NWK * NCH1 + wid], S[3])
        pltpu.sync_copy(dst2.at[NWK * NCH1 + wid], D[3])
        xb = pl.multiple_of(XBASE + wid * 128, 8)
        pltpu.async_copy(xl_hbm.at[S[3]], bL[1], semG)
        pltpu.async_copy(xr_hbm.at[D[3]], bR[1], semG)
        pltpu.async_copy(e_hbm.at[pl.ds(xb, 128)], bE[1], semG)
        drain_gather(1)
        compute(1)
        pltpu.sync_copy(bM[1], msg_out.at[pl.ds(xb, 128)])


@functools.partial(
    pl.kernel,
    out_type=jax.ShapeDtypeStruct((6, NN, DE), _F32),
    mesh=_MESH,
    compiler_params=pltpu.CompilerParams(needs_layout_passes=False,
                                         use_tc_tiling_on_sc=False),
    scratch_types=[
        pltpu.VMEM((128,), _I32), pltpu.VMEM((128,), _I32),
        pltpu.VMEM((128,), _I32), pltpu.VMEM((128,), _I32),
        pltpu.VMEM((128, DE), _F32), pltpu.VMEM((128, DE), _F32),
        pltpu.VMEM((128, DE), _F32), pltpu.VMEM((128, DE), _F32),
        pltpu.VMEM((_ZR, DE), _F32),
        pltpu.VMEM_SHARED((NN, DE), _F32),
        pltpu.SemaphoreType.DMA,
        pltpu.SemaphoreType.DMA,
        pltpu.SemaphoreType.DMA,
    ],
)
def _pass2(dst2, msg_hbm, acc_out,
           S0, S1, S2, S3, B0, B1, B2, B3, zb, accS, semI, semG, semS):
    cid = lax.axis_index("c")
    sid = lax.axis_index("s")
    S = [S0, S1, S2, S3]
    B = [B0, B1, B2, B3]
    _fill_vec(zb, _ZR, DE, jnp.zeros((16,), _F32))

    def run_phase(q, qout, row_base, eb_base, nch, n_extra, extra_row,
                  extra_eb):
        _zero_spmem(accS, zb, sid)
        plsc.subcore_barrier()

        def fire_idx(x, j):
            pltpu.async_copy(dst2.at[row_base + x], S[j], semI)

        def drain_idx():
            pltpu.make_async_copy(dst2.at[0], S[0], semI).wait()

        def fire_read(x, p):
            b = pl.multiple_of(eb_base + x * 128, 8)
            pltpu.async_copy(msg_hbm.at[pl.ds(b, 128), q], B[p], semG)

        def drain_read(p):
            pltpu.make_async_copy(msg_hbm.at[pl.ds(0, 128), q], B[p],
                                  semG).wait()

        def fire_scat(x, j, p):
            pltpu.async_copy(B[p], accS.at[S[j]], semS, add=True)

        def drain_scat():
            pltpu.make_async_copy(B[0], accS.at[S[0]], semS).wait()

        fire_idx(0, 0)
        fire_idx(1, 1)
        drain_idx()
        fire_read(0, 0)

        def body(k4, _):
            x0 = k4 * 4
            for j in range(4):
                x = x0 + j

                @pl.when(x >= 1)
                def _():
                    drain_scat()
                fire_idx(x + 2, (j + 2) % 4)
                drain_idx()
                fire_read(x + 1, (j + 1) % 4)
                drain_read(j)
                fire_scat(x, j, j)
            return 0
        nbody = (nch - 2) // 4
        lax.fori_loop(0, nbody, body, 0)

        for x in range(nbody * 4, nch):
            j = x % 4
            drain_scat()
            if x + 2 < nch:
                fire_idx(x + 2, (j + 2) % 4)
            if x + 1 < nch:
                drain_idx()
                fire_read(x + 1, (j + 1) % 4)
            drain_read(j)
            fire_scat(x, j, j)
        drain_scat()

        @pl.when(sid < n_extra)
        def _():
            pltpu.sync_copy(dst2.at[extra_row], S[0])
            xb = pl.multiple_of(extra_eb, 8)
            pltpu.sync_copy(msg_hbm.at[pl.ds(xb, 128), q], B[0])
            pltpu.sync_copy(B[0], accS.at[S[0]], add=True)

        plsc.subcore_barrier()
        _copy_out_rows(accS, acc_out.at[qout], sid)
        plsc.subcore_barrier()

    for ph in range(2):
        q = cid * 2 + ph
        run_phase(q, q,
                  sid * 390, sid * (390 * 128), 390,
                  10, NWK * NCH1 + sid, XBASE + sid * 128)

    run_phase(4, 4 + cid,
              cid * 3125 + sid * 195,
              cid * 400000 + sid * (195 * 128), 195,
              5, cid * 3125 + 3120 + sid,
              cid * 400000 + 399360 + sid * 128)


_RB = 1000
_REB = 2000


def _prep_body(x, wpt, bp, lng, lnb, wlt, bl, wrt, br, wet, attr, sl, sd,
               xl_o, xr_o, ex_o, lm_o):
    h = jnp.dot(x[...], wpt[...], preferred_element_type=_F32) + bp[...]
    mu = jnp.mean(h, axis=-1, keepdims=True)
    var = jnp.mean((h - mu) ** 2, axis=-1, keepdims=True)
    hn = (h - mu) * lax.rsqrt(var + 1e-5) * lng[...] + lnb[...]
    xl = jnp.dot(hn, wlt[...], preferred_element_type=_F32) + bl[...]
    xr = jnp.dot(hn, wrt[...], preferred_element_type=_F32) + br[...]
    deg = sd[...][0, :, 0:1] + sd[...][1, :, 0:1]
    lm = (sl[...][0] + sl[...][1]) / jnp.maximum(deg, 1.0)
    el = jnp.dot(lm, wet[...], preferred_element_type=_F32)
    z = xl + xr + el
    z = jnp.maximum(z, NEG * z)
    logit = jnp.sum(z * attr[...], axis=-1, keepdims=True)
    xl_o[...] = xl
    xr_o[...] = xr
    ex_o[...] = jnp.exp(logit)
    lm_o[...] = lm


def _finish_prep_body(acc, exs, xl, bias, lng, lnb, wlt, bl, wrt, br,
                      wet, attr, lm, xl_o, xr_o, ex_o):
    a = acc[...]
    num = jnp.concatenate([a[0], a[1], a[2], a[3]], axis=-1) + exs[...] * xl[...]
    d = a[4][:, 0:1] + a[5][:, 0:1] + exs[...] + 1e-16
    h = jnp.maximum(num / d + bias[...], 0.0)
    mu = jnp.mean(h, axis=-1, keepdims=True)
    var = jnp.mean((h - mu) ** 2, axis=-1, keepdims=True)
    hn = (h - mu) * lax.rsqrt(var + 1e-5) * lng[...] + lnb[...]
    xl2 = jnp.dot(hn, wlt[...], preferred_element_type=_F32) + bl[...]
    xr2 = jnp.dot(hn, wrt[...], preferred_element_type=_F32) + br[...]
    el = jnp.dot(lm[...], wet[...], preferred_element_type=_F32)
    z = xl2 + xr2 + el
    z = jnp.maximum(z, NEG * z)
    logit = jnp.sum(z * attr[...], axis=-1, keepdims=True)
    xl_o[...] = xl2
    xr_o[...] = xr2
    ex_o[...] = jnp.exp(logit)


def _finish_cls_body(acc, exs, xl, bias, wct, bc, cls_o, h_o):
    a = acc[...]
    num = jnp.concatenate([a[0], a[1], a[2], a[3]], axis=-1) + exs[...] * xl[...]
    d = a[4][:, 0:1] + a[5][:, 0:1] + exs[...] + 1e-16
    h = jnp.maximum(num / d + bias[...], 0.0)
    h_o[...] = h
    cls_o[...] = jnp.dot(h, wct[...], preferred_element_type=_F32) + bc[...]


def _edge_e_body(ea, w1, w2, e1_o, e2_o):
    a = ea[...]
    e1_o[...] = jnp.dot(a, w1[...], preferred_element_type=_F32)
    e2_o[...] = jnp.dot(a, w2[...], preferred_element_type=_F32)


def _row_spec(shape):
    nd = len(shape)
    if nd == 2:
        return pl.BlockSpec((_RB, shape[1]), lambda i: (i, 0))
    return pl.BlockSpec((shape[0], _RB, shape[2]), lambda i: (0, i, 0))


def _const_spec(shape):
    return pl.BlockSpec(shape, lambda i: tuple(0 for _ in shape))


def _tc_edge_e(ea, wet1, wet2):
    return pl.pallas_call(
        _edge_e_body,
        grid=(EE // _REB,),
        in_specs=[pl.BlockSpec((_REB, DE), lambda i: (i, 0)),
                  _const_spec((DE, HH)), _const_spec((DE, HH))],
        out_specs=[pl.BlockSpec((_REB, HH), lambda i: (i, 0)),
                   pl.BlockSpec((_REB, HH), lambda i: (i, 0))],
        out_shape=[jax.ShapeDtypeStruct((EE, HH), _F32),
                   jax.ShapeDtypeStruct((EE, HH), _F32)],
    )(ea, wet1, wet2)


def _tc_prep(x, wpt, bp, lng, lnb, wlt, bl, wrt, br, wet, attr, sumL, sumD):
    return pl.pallas_call(
        _prep_body,
        grid=(NN // _RB,),
        in_specs=[pl.BlockSpec((_RB, DIN), lambda i: (i, 0)),
                  _const_spec((DIN, HH)), _const_spec((1, HH)),
                  _const_spec((1, HH)), _const_spec((1, HH)),
                  _const_spec((HH, HH)), _const_spec((1, HH)),
                  _const_spec((HH, HH)), _const_spec((1, HH)),
                  _const_spec((DE, HH)), _const_spec((1, HH)),
                  _row_spec((NC, NN, DE)), _row_spec((NC, NN, DE))],
        out_specs=[pl.BlockSpec((_RB, HH), lambda i: (i, 0)),
                   pl.BlockSpec((_RB, HH), lambda i: (i, 0)),
                   pl.BlockSpec((_RB, 1), lambda i: (i, 0)),
                   pl.BlockSpec((_RB, DE), lambda i: (i, 0))],
        out_shape=[jax.ShapeDtypeStruct((NN, HH), _F32),
                   jax.ShapeDtypeStruct((NN, HH), _F32),
                   jax.ShapeDtypeStruct((NN, 1), _F32),
                   jax.ShapeDtypeStruct((NN, DE), _F32)],
    )(x, wpt, bp, lng, lnb, wlt, bl, wrt, br, wet, attr, sumL, sumD)


def _tc_finish_prep(acc, exs, xl, bias, lng, lnb, wlt, bl, wrt, br,
                    wet, attr, lm):
    return pl.pallas_call(
        _finish_prep_body,
        grid=(NN // _RB,),
        in_specs=[_row_spec((6, NN, DE)),
                  pl.BlockSpec((_RB, 1), lambda i: (i, 0)),
                  pl.BlockSpec((_RB, HH), lambda i: (i, 0)),
                  _const_spec((1, HH)),
                  _const_spec((1, HH)), _const_spec((1, HH)),
                  _const_spec((HH, HH)), _const_spec((1, HH)),
                  _const_spec((HH, HH)), _const_spec((1, HH)),
                  _const_spec((DE, HH)), _const_spec((1, HH)),
                  pl.BlockSpec((_RB, DE), lambda i: (i, 0))],
        out_specs=[pl.BlockSpec((_RB, HH), lambda i: (i, 0)),
                   pl.BlockSpec((_RB, HH), lambda i: (i, 0)),
                   pl.BlockSpec((_RB, 1), lambda i: (i, 0))],
        out_shape=[jax.ShapeDtypeStruct((NN, HH), _F32),
                   jax.ShapeDtypeStruct((NN, HH), _F32),
                   jax.ShapeDtypeStruct((NN, 1), _F32)],
    )(acc, exs, xl, bias, lng, lnb, wlt, bl, wrt, br, wet, attr, lm)


def _tc_finish_cls(acc, exs, xl, bias, wct, bc):
    return pl.pallas_call(
        _finish_cls_body,
        grid=(NN // _RB,),
        in_specs=[_row_spec((6, NN, DE)),
                  pl.BlockSpec((_RB, 1), lambda i: (i, 0)),
                  pl.BlockSpec((_RB, HH), lambda i: (i, 0)),
                  _const_spec((1, HH)),
                  _const_spec((HH, 64)), _const_spec((1, 64))],
        out_specs=[pl.BlockSpec((_RB, 64), lambda i: (i, 0)),
                   pl.BlockSpec((_RB, HH), lambda i: (i, 0))],
        out_shape=[jax.ShapeDtypeStruct((NN, 64), _F32),
                   jax.ShapeDtypeStruct((NN, HH), _F32)],
    )(acc, exs, xl, bias, wct, bc)


def kernel(x, edge_index, edge_attr, params):
    src = edge_index[0]
    dst = edge_index[1]
    src2 = src.reshape(NROW, 128)
    dst2 = dst.reshape(NROW, 128)
    p1, p2 = params['layers']

    wpt = params['Wp'].T
    bp = params['bp'].reshape(1, HH)
    wct = params['Wc'].T
    bc = params['bc'].reshape(1, 64)

    def layer_mats(p):
        return (p['ln_g'].reshape(1, HH), p['ln_b'].reshape(1, HH),
                p['Wl'].T, p['bl'].reshape(1, HH),
                p['Wr'].T, p['br'].reshape(1, HH),
                p['We'].T, p['att'].reshape(1, HH),
                jnp.broadcast_to(p['att'].reshape(HH, 1), (HH, 16)),
                p['bias'].reshape(1, HH))

    (lng1, lnb1, wlt1, bl1, wrt1, br1, wet1, attr1, attv1, bias1) = layer_mats(p1)
    (lng2, lnb2, wlt2, bl2, wrt2, br2, wet2, attr2, attv2, bias2) = layer_mats(p2)

    sumL, sumD = _pass0(dst, edge_attr)
    e1, e2 = _tc_edge_e(edge_attr, wet1, wet2)
    xl1, xr1, exs1, lm = _tc_prep(x, wpt, bp, lng1, lnb1, wlt1, bl1, wrt1,
                                  br1, wet1, attr1, sumL, sumD)
    msg1 = _pass1(src2, dst2, xl1, xr1, e1, attv1)
    acc1 = _pass2(dst2, msg1)
    xl2, xr2, exs2 = _tc_finish_prep(acc1, exs1, xl1, bias1, lng2,
                                     lnb2, wlt2, bl2, wrt2, br2, wet2,
                                     attr2, lm)
    msg2 = _pass1(src2, dst2, xl2, xr2, e2, attv2)
    acc2 = _pass2(dst2, msg2)
    cls, h = _tc_finish_cls(acc2, exs2, xl2, bias2, wct, bc)
    return (cls, h)


# speedup vs baseline: 1.4505x; 1.0001x over previous
import functools

import jax
import jax.numpy as jnp
from jax import lax
from jax.experimental import pallas as pl
from jax.experimental.pallas import tpu as pltpu
from jax.experimental.pallas import tpu_sc as plsc

NN = 50000
EE = 800000
DIN = 128
DE = 16
HH = 64
NEG = 0.2
NC = 2
NS = 16
NWK = NC * NS
EPT = EE // NWK

NCH1 = 195
EB1 = NCH1 * 128
XBASE = NWK * EB1
NROW = EE // 128

_MESH = plsc.VectorSubcoreMesh(core_axis_name="c", subcore_axis_name="s")

_F32 = jnp.float32
_I32 = jnp.int32

_CP = 3128
_CP_LAST = NN - (NS - 1) * _CP
_ZR = 200


def _fill_vec(ref, rows, cols, vec):
    nslot = cols // 16

    def body(i, _):
        r = i // nslot
        s = i % nslot
        ref[r, pl.ds(s * 16, 16)] = vec
        return 0

    lax.fori_loop(0, rows * nslot, body, 0)


def _zero_idx(ref, n):
    z = jnp.zeros((16,), _I32)
    for i in range(n // 16):
        ref[pl.ds(i * 16, 16)] = z


def _zero_spmem(acc, zb, sid):
    nchunk = NN // _ZR

    def body(j, _):
        c = sid + NS * j

        @pl.when(c < nchunk)
        def _():
            r0 = pl.multiple_of(c * _ZR, 8)
            pltpu.sync_copy(zb, acc.at[pl.ds(r0, _ZR)])
        return 0

    lax.fori_loop(0, (nchunk + NS - 1) // NS, body, 0)


def _copy_out_rows(acc, out_at, sid):
    @pl.when(sid < NS - 1)
    def _():
        r0 = pl.multiple_of(sid * _CP, 8)
        pltpu.sync_copy(acc.at[pl.ds(r0, _CP)], out_at.at[pl.ds(r0, _CP)])

    @pl.when(sid == NS - 1)
    def _():
        r0 = (NS - 1) * _CP
        pltpu.sync_copy(acc.at[pl.ds(r0, _CP_LAST)],
                        out_at.at[pl.ds(r0, _CP_LAST)])


@functools.partial(
    pl.kernel,
    out_type=(
        jax.ShapeDtypeStruct((NC, NN, DE), _F32),
        jax.ShapeDtypeStruct((NC, NN, DE), _F32),
    ),
    mesh=_MESH,
    compiler_params=pltpu.CompilerParams(needs_layout_passes=False,
                                         use_tc_tiling_on_sc=False),
    scratch_types=[
        pltpu.VMEM((128,), _I32),
        pltpu.VMEM((48,), _I32),
        pltpu.VMEM((128, DE), _F32),
        pltpu.VMEM((48, DE), _F32),
        pltpu.VMEM((_ZR, DE), _F32),
        pltpu.VMEM_SHARED((NN, DE), _F32),
    ],
)
def _pass0(dst_hbm, ea_hbm, outL, outD, idx, idxt, ea, eat, zb, acc):
    cid = lax.axis_index("c")
    sid = lax.axis_index("s")
    wid = sid * NC + cid
    base0 = wid * EPT
    zv = jnp.zeros((16,), _F32)
    ov = jnp.ones((16,), _F32)

    _fill_vec(zb, _ZR, DE, zv)
    _zero_idx(idxt, 48)
    pltpu.sync_copy(dst_hbm.at[pl.ds(base0 + 195 * 128, 40)],
                    idxt.at[pl.ds(0, 40)])

    _zero_spmem(acc, zb, sid)
    plsc.subcore_barrier()

    def chunk1(k, _):
        b = base0 + k * 128
        pltpu.sync_copy(dst_hbm.at[pl.ds(b, 128)], idx)
        pltpu.sync_copy(ea_hbm.at[pl.ds(b, 128)], ea)
        pltpu.sync_copy(ea, acc.at[idx], add=True)
        return 0
    lax.fori_loop(0, 195, chunk1, 0)

    _fill_vec(eat, 48, DE, zv)
    pltpu.sync_copy(ea_hbm.at[pl.ds(base0 + 195 * 128, 40)],
                    eat.at[pl.ds(0, 40)])
    pltpu.sync_copy(eat, acc.at[idxt], add=True)

    plsc.subcore_barrier()
    _copy_out_rows(acc, outL.at[cid], sid)
    plsc.subcore_barrier()

    _zero_spmem(acc, zb, sid)
    plsc.subcore_barrier()

    _fill_vec(ea, 128, DE, ov)

    def chunk2(k, _):
        b = base0 + k * 128
        pltpu.sync_copy(dst_hbm.at[pl.ds(b, 128)], idx)
        pltpu.sync_copy(ea, acc.at[idx], add=True)
        return 0
    lax.fori_loop(0, 195, chunk2, 0)

    _fill_vec(eat, 40, DE, ov)
    pltpu.sync_copy(eat, acc.at[idxt], add=True)

    plsc.subcore_barrier()
    _copy_out_rows(acc, outD.at[cid], sid)


@functools.partial(
    pl.kernel,
    out_type=jax.ShapeDtypeStruct((EE, 5, DE), _F32),
    mesh=_MESH,
    compiler_params=pltpu.CompilerParams(needs_layout_passes=False,
                                         use_tc_tiling_on_sc=False),
    scratch_types=[
        pltpu.VMEM((128,), _I32), pltpu.VMEM((128,), _I32),
        pltpu.VMEM((128,), _I32), pltpu.VMEM((128,), _I32),
        pltpu.VMEM((128,), _I32), pltpu.VMEM((128,), _I32),
        pltpu.VMEM((128,), _I32), pltpu.VMEM((128,), _I32),
        pltpu.VMEM((128, HH), _F32), pltpu.VMEM((128, HH), _F32),
        pltpu.VMEM((128, HH), _F32), pltpu.VMEM((128, HH), _F32),
        pltpu.VMEM((128, HH), _F32), pltpu.VMEM((128, HH), _F32),
        pltpu.VMEM((128, 5, DE), _F32), pltpu.VMEM((128, 5, DE), _F32),
        pltpu.VMEM((HH, 16), _F32),
        pltpu.SemaphoreType.DMA,
        pltpu.SemaphoreType.DMA,
        pltpu.SemaphoreType.DMA,
    ],
)
def _pass1(src2, dst2, xl_hbm, xr_hbm, e_hbm, att_hbm, msg_out,
           S0, S1, S2, S3, D0, D1, D2, D3,
           bL0, bL1, bR0, bR1, bE0, bE1, bM0, bM1,
           attb, semI, semG, semO):
    cid = lax.axis_index("c")
    sid = lax.axis_index("s")
    wid = sid * NC + cid
    row0 = wid * NCH1
    eb0 = wid * EB1
    S = [S0, S1, S2, S3]
    D = [D0, D1, D2, D3]
    bL = [bL0, bL1]
    bR = [bR0, bR1]
    bE = [bE0, bE1]
    bM = [bM0, bM1]

    pltpu.sync_copy(att_hbm, attb)
    iot = lax.iota(_I32, 16)
    rows_list = [iot + g * 16 for g in range(8)]

    def fire_idx(x, j):
        pltpu.async_copy(src2.at[row0 + x], S[j], semI)
        pltpu.async_copy(dst2.at[row0 + x], D[j], semI)

    def drain_idx():
        pltpu.make_async_copy(src2.at[0], S[0], semI).wait()
        pltpu.make_async_copy(dst2.at[0], D[0], semI).wait()

    def fire_gather(x, j, p):
        b = pl.multiple_of(eb0 + x * 128, 8)
        for hh in range(4):
            sl = pl.ds(hh * 32, 32)
            pltpu.async_copy(xl_hbm.at[S[j].at[sl]], bL[p].at[sl], semG)
            pltpu.async_copy(xr_hbm.at[D[j].at[sl]], bR[p].at[sl], semG)
        pltpu.async_copy(e_hbm.at[pl.ds(b, 128)], bE[p], semG)

    def drain_gather(p):
        for hh in range(4):
            sl = pl.ds(hh * 32, 32)
            pltpu.make_async_copy(xl_hbm.at[S[0].at[sl]], bL[p].at[sl],
                                  semG).wait()
            pltpu.make_async_copy(xr_hbm.at[D[0].at[sl]], bR[p].at[sl],
                                  semG).wait()
        pltpu.make_async_copy(e_hbm.at[pl.ds(0, 128)], bE[p], semG).wait()

    def fire_out(x, p):
        b = pl.multiple_of(eb0 + x * 128, 8)
        pltpu.async_copy(bM[p], msg_out.at[pl.ds(b, 128)], semO)

    def drain_out(p):
        pltpu.make_async_copy(bM[p], msg_out.at[pl.ds(0, 128)], semO).wait()

    def compute(p):
        bLp, bRp, bEp, bMp = bL[p], bR[p], bE[p], bM[p]

        def dbody(d, accs):
            dv = jnp.full((16,), d, _I32)
            ad = plsc.load_gather(attb, [dv, iot])
            new = []
            for g in range(8):
                rows = rows_list[g]
                xld = plsc.load_gather(bLp, [rows, dv])
                xrd = plsc.load_gather(bRp, [rows, dv])
                ed = plsc.load_gather(bEp, [rows, dv])
                z = xld + xrd + ed
                z = jnp.maximum(z, NEG * z)
                new.append(accs[g] + ad * z)
            return tuple(new)

        accs = lax.fori_loop(0, HH, dbody,
                             tuple(jnp.zeros((16,), _F32) for _ in range(8)))
        exs = [jnp.exp(a) for a in accs]

        def mbody(d, _):
            dv = jnp.full((16,), d, _I32)
            for q in range(4):
                qv = jnp.full((16,), q, _I32)
                cv = dv + q * DE
                for g in range(8):
                    rows = rows_list[g]
                    xld = plsc.load_gather(bLp, [rows, cv])
                    plsc.store_scatter(bMp, [rows, qv, dv], exs[g] * xld)
            return 0
        lax.fori_loop(0, DE, mbody, 0)

        qv4 = jnp.full((16,), 4, _I32)

        def xbody(d, _):
            dv = jnp.full((16,), d, _I32)
            for g in range(8):
                plsc.store_scatter(bMp, [rows_list[g], qv4, dv], exs[g])
            return 0
        lax.fori_loop(0, DE, xbody, 0)

    fire_idx(0, 0)
    fire_idx(1, 1)
    drain_idx()
    fire_gather(0, 0, 0)

    def body(k4, _):
        x0 = k4 * 4
        for j in range(4):
            x = x0 + j
            p = j % 2
            fire_idx(x + 2, (j + 2) % 4)
            drain_idx()
            fire_gather(x + 1, (j + 1) % 4, (p + 1) % 2)
            drain_gather(p)

            @pl.when(x >= 2)
            def _():
                drain_out(p)
            compute(p)
            fire_out(x, p)
        return 0
    lax.fori_loop(0, (NCH1 - 3) // 4, body, 0)

    fire_idx(194, 2)
    drain_idx()
    fire_gather(193, 1, 1)
    drain_gather(0)
    drain_out(0)
    compute(0)
    fire_out(192, 0)

    drain_idx()
    fire_gather(194, 2, 0)
    drain_gather(1)
    drain_out(1)
    compute(1)
    fire_out(193, 1)

    drain_gather(0)
    drain_out(0)
    compute(0)
    fire_out(194, 0)

    drain_out(1)
    drain_out(0)

    @pl.when(wid < 10)
    def _():
        pltpu.sync_copy(src2.at[NWK * NCH1 + wid], S[3])
        pltpu.sync_copy(dst2.at[NWK * NCH1 + wid], D[3])
        xb = pl.multiple_of(XBASE + wid * 128, 8)
        pltpu.async_copy(xl_hbm.at[S[3]], bL[1], semG)
        pltpu.async_copy(xr_hbm.at[D[3]], bR[1], semG)
        pltpu.async_copy(e_hbm.at[pl.ds(xb, 128)], bE[1], semG)
        drain_gather(1)
        compute(1)
        pltpu.sync_copy(bM[1], msg_out.at[pl.ds(xb, 128)])


@functools.partial(
    pl.kernel,
    out_type=jax.ShapeDtypeStruct((6, NN, DE), _F32),
    mesh=_MESH,
    compiler_params=pltpu.CompilerParams(needs_layout_passes=False,
                                         use_tc_tiling_on_sc=False),
    scratch_types=[
        pltpu.VMEM((128,), _I32), pltpu.VMEM((128,), _I32),
        pltpu.VMEM((128,), _I32), pltpu.VMEM((128,), _I32),
        pltpu.VMEM((128, DE), _F32), pltpu.VMEM((128, DE), _F32),
        pltpu.VMEM((128, DE), _F32), pltpu.VMEM((128, DE), _F32),
        pltpu.VMEM((_ZR, DE), _F32),
        pltpu.VMEM_SHARED((NN, DE), _F32),
        pltpu.SemaphoreType.DMA,
        pltpu.SemaphoreType.DMA,
        pltpu.SemaphoreType.DMA,
    ],
)
def _pass2(dst2, msg_hbm, acc_out,
           S0, S1, S2, S3, B0, B1, B2, B3, zb, accS, semI, semG, semS):
    cid = lax.axis_index("c")
    sid = lax.axis_index("s")
    S = [S0, S1, S2, S3]
    B = [B0, B1, B2, B3]
    _fill_vec(zb, _ZR, DE, jnp.zeros((16,), _F32))

    def run_phase(q, qout, row_base, eb_base, nch, n_extra, extra_row,
                  extra_eb):
        _zero_spmem(accS, zb, sid)
        plsc.subcore_barrier()

        def fire_idx(x, j):
            pltpu.async_copy(dst2.at[row_base + x], S[j], semI)

        def drain_idx():
            pltpu.make_async_copy(dst2.at[0], S[0], semI).wait()

        def fire_read(x, p):
            b = pl.multiple_of(eb_base + x * 128, 8)
            pltpu.async_copy(msg_hbm.at[pl.ds(b, 128), q], B[p], semG)

        def drain_read(p):
            pltpu.make_async_copy(msg_hbm.at[pl.ds(0, 128), q], B[p],
                                  semG).wait()

        def fire_scat(x, j, p):
            pltpu.async_copy(B[p], accS.at[S[j]], semS, add=True)

        def drain_scat():
            pltpu.make_async_copy(B[0], accS.at[S[0]], semS).wait()

        fire_idx(0, 0)
        fire_idx(1, 1)
        drain_idx()
        fire_read(0, 0)

        def body(k4, _):
            x0 = k4 * 4
            for j in range(4):
                x = x0 + j

                @pl.when(x >= 1)
                def _():
                    drain_scat()
                fire_idx(x + 2, (j + 2) % 4)
                drain_idx()
                fire_read(x + 1, (j + 1) % 4)
                drain_read(j)
                fire_scat(x, j, j)
            return 0
        nbody = (nch - 2) // 4
        lax.fori_loop(0, nbody, body, 0)

        for x in range(nbody * 4, nch):
            j = x % 4
            drain_scat()
            if x + 2 < nch:
                fire_idx(x + 2, (j + 2) % 4)
            if x + 1 < nch:
                drain_idx()
                fire_read(x + 1, (j + 1) % 4)
            drain_read(j)
            fire_scat(x, j, j)
        drain_scat()

        @pl.when(sid < n_extra)
        def _():
            pltpu.sync_copy(dst2.at[extra_row], S[0])
            xb = pl.multiple_of(extra_eb, 8)
            pltpu.sync_copy(msg_hbm.at[pl.ds(xb, 128), q], B[0])
            pltpu.sync_copy(B[0], accS.at[S[0]], add=True)

        plsc.subcore_barrier()
        _copy_out_rows(accS, acc_out.at[qout], sid)
        plsc.subcore_barrier()

    for ph in range(2):
        q = cid * 2 + ph
        run_phase(q, q,
                  sid * 390, sid * (390 * 128), 390,
                  10, NWK * NCH1 + sid, XBASE + sid * 128)

    run_phase(4, 4 + cid,
              cid * 3125 + sid * 195,
              cid * 400000 + sid * (195 * 128), 195,
              5, cid * 3125 + 3120 + sid,
              cid * 400000 + 399360 + sid * 128)


_RB = 1000
_REB = 2000


def _prep_body(x, wpt, bp, lng, lnb, wlt, bl, wrt, br, wet, attr, sl, sd,
               xl_o, xr_o, ex_o, lm_o):
    h = jnp.dot(x[...], wpt[...], preferred_element_type=_F32) + bp[...]
    mu = jnp.mean(h, axis=-1, keepdims=True)
    var = jnp.mean((h - mu) ** 2, axis=-1, keepdims=True)
    hn = (h - mu) * lax.rsqrt(var + 1e-5) * lng[...] + lnb[...]
    xl = jnp.dot(hn, wlt[...], preferred_element_type=_F32) + bl[...]
    xr = jnp.dot(hn, wrt[...], preferred_element_type=_F32) + br[...]
    deg = sd[...][0, :, 0:1] + sd[...][1, :, 0:1]
    lm = (sl[...][0] + sl[...][1]) / jnp.maximum(deg, 1.0)
    el = jnp.dot(lm, wet[...], preferred_element_type=_F32)
    z = xl + xr + el
    z = jnp.maximum(z, NEG * z)
    logit = jnp.sum(z * attr[...], axis=-1, keepdims=True)
    xl_o[...] = xl
    xr_o[...] = xr
    ex_o[...] = jnp.exp(logit)
    lm_o[...] = lm


def _finish_prep_body(acc, exs, xl, bias, lng, lnb, wlt, bl, wrt, br,
                      wet, attr, lm, xl_o, xr_o, ex_o):
    a = acc[...]
    num = jnp.concatenate([a[0], a[1], a[2], a[3]], axis=-1) + exs[...] * xl[...]
    d = a[4][:, 0:1] + a[5][:, 0:1] + exs[...] + 1e-16
    h = jnp.maximum(num / d + bias[...], 0.0)
    mu = jnp.mean(h, axis=-1, keepdims=True)
    var = jnp.mean((h - mu) ** 2, axis=-1, keepdims=True)
    hn = (h - mu) * lax.rsqrt(var + 1e-5) * lng[...] + lnb[...]
    xl2 = jnp.dot(hn, wlt[...], preferred_element_type=_F32) + bl[...]
    xr2 = jnp.dot(hn, wrt[...], preferred_element_type=_F32) + br[...]
    el = jnp.dot(lm[...], wet[...], preferred_element_type=_F32)
    z = xl2 + xr2 + el
    z = jnp.maximum(z, NEG * z)
    logit = jnp.sum(z * attr[...], axis=-1, keepdims=True)
    xl_o[...] = xl2
    xr_o[...] = xr2
    ex_o[...] = jnp.exp(logit)


def _finish_cls_body(acc, exs, xl, bias, wct, bc, cls_o, h_o):
    a = acc[...]
    num = jnp.concatenate([a[0], a[1], a[2], a[3]], axis=-1) + exs[...] * xl[...]
    d = a[4][:, 0:1] + a[5][:, 0:1] + exs[...] + 1e-16
    h = jnp.maximum(num / d + bias[...], 0.0)
    h_o[...] = h
    cls_o[...] = jnp.dot(h, wct[...], preferred_element_type=_F32) + bc[...]


def _edge_e_body(ea, w1, w2, e1_o, e2_o):
    a = ea[...]
    e1_o[...] = jnp.dot(a, w1[...], preferred_element_type=_F32)
    e2_o[...] = jnp.dot(a, w2[...], preferred_element_type=_F32)


def _row_spec(shape):
    nd = len(shape)
    if nd == 2:
        return pl.BlockSpec((_RB, shape[1]), lambda i: (i, 0))
    return pl.BlockSpec((shape[0], _RB, shape[2]), lambda i: (0, i, 0))


def _const_spec(shape):
    return pl.BlockSpec(shape, lambda i: tuple(0 for _ in shape))


def _tc_edge_e(ea, wet1, wet2):
    return pl.pallas_call(
        _edge_e_body,
        grid=(EE // _REB,),
        in_specs=[pl.BlockSpec((_REB, DE), lambda i: (i, 0)),
                  _const_spec((DE, HH)), _const_spec((DE, HH))],
        out_specs=[pl.BlockSpec((_REB, HH), lambda i: (i, 0)),
                   pl.BlockSpec((_REB, HH), lambda i: (i, 0))],
        out_shape=[jax.ShapeDtypeStruct((EE, HH), _F32),
                   jax.ShapeDtypeStruct((EE, HH), _F32)],
    )(ea, wet1, wet2)


def _tc_prep(x, wpt, bp, lng, lnb, wlt, bl, wrt, br, wet, attr, sumL, sumD):
    return pl.pallas_call(
        _prep_body,
        grid=(NN // _RB,),
        in_specs=[pl.BlockSpec((_RB, DIN), lambda i: (i, 0)),
                  _const_spec((DIN, HH)), _const_spec((1, HH)),
                  _const_spec((1, HH)), _const_spec((1, HH)),
                  _const_spec((HH, HH)), _const_spec((1, HH)),
                  _const_spec((HH, HH)), _const_spec((1, HH)),
                  _const_spec((DE, HH)), _const_spec((1, HH)),
                  _row_spec((NC, NN, DE)), _row_spec((NC, NN, DE))],
        out_specs=[pl.BlockSpec((_RB, HH), lambda i: (i, 0)),
                   pl.BlockSpec((_RB, HH), lambda i: (i, 0)),
                   pl.BlockSpec((_RB, 1), lambda i: (i, 0)),
                   pl.BlockSpec((_RB, DE), lambda i: (i, 0))],
        out_shape=[jax.ShapeDtypeStruct((NN, HH), _F32),
                   jax.ShapeDtypeStruct((NN, HH), _F32),
                   jax.ShapeDtypeStruct((NN, 1), _F32),
                   jax.ShapeDtypeStruct((NN, DE), _F32)],
    )(x, wpt, bp, lng, lnb, wlt, bl, wrt, br, wet, attr, sumL, sumD)


def _tc_finish_prep(acc, exs, xl, bias, lng, lnb, wlt, bl, wrt, br,
                    wet, attr, lm):
    return pl.pallas_call(
        _finish_prep_body,
        grid=(NN // _RB,),
        in_specs=[_row_spec((6, NN, DE)),
                  pl.BlockSpec((_RB, 1), lambda i: (i, 0)),
                  pl.BlockSpec((_RB, HH), lambda i: (i, 0)),
                  _const_spec((1, HH)),
                  _const_spec((1, HH)), _const_spec((1, HH)),
                  _const_spec((HH, HH)), _const_spec((1, HH)),
                  _const_spec((HH, HH)), _const_spec((1, HH)),
                  _const_spec((DE, HH)), _const_spec((1, HH)),
                  pl.BlockSpec((_RB, DE), lambda i: (i, 0))],
        out_specs=[pl.BlockSpec((_RB, HH), lambda i: (i, 0)),
                   pl.BlockSpec((_RB, HH), lambda i: (i, 0)),
                   pl.BlockSpec((_RB, 1), lambda i: (i, 0))],
        out_shape=[jax.ShapeDtypeStruct((NN, HH), _F32),
                   jax.ShapeDtypeStruct((NN, HH), _F32),
                   jax.ShapeDtypeStruct((NN, 1), _F32)],
    )(acc, exs, xl, bias, lng, lnb, wlt, bl, wrt, br, wet, attr, lm)


def _tc_finish_cls(acc, exs, xl, bias, wct, bc):
    return pl.pallas_call(
        _finish_cls_body,
        grid=(NN // _RB,),
        in_specs=[_row_spec((6, NN, DE)),
                  pl.BlockSpec((_RB, 1), lambda i: (i, 0)),
                  pl.BlockSpec((_RB, HH), lambda i: (i, 0)),
                  _const_spec((1, HH)),
                  _const_spec((HH, 64)), _const_spec((1, 64))],
        out_specs=[pl.BlockSpec((_RB, 64), lambda i: (i, 0)),
                   pl.BlockSpec((_RB, HH), lambda i: (i, 0))],
        out_shape=[jax.ShapeDtypeStruct((NN, 64), _F32),
                   jax.ShapeDtypeStruct((NN, HH), _F32)],
    )(acc, exs, xl, bias, wct, bc)


def kernel(x, edge_index, edge_attr, params):
    src = edge_index[0]
    dst = edge_index[1]
    src2 = src.reshape(NROW, 128)
    dst2 = dst.reshape(NROW, 128)
    p1, p2 = params['layers']

    wpt = params['Wp'].T
    bp = params['bp'].reshape(1, HH)
    wct = params['Wc'].T
    bc = params['bc'].reshape(1, 64)

    def layer_mats(p):
        return (p['ln_g'].reshape(1, HH), p['ln_b'].reshape(1, HH),
                p['Wl'].T, p['bl'].reshape(1, HH),
                p['Wr'].T, p['br'].reshape(1, HH),
                p['We'].T, p['att'].reshape(1, HH),
                jnp.broadcast_to(p['att'].reshape(HH, 1), (HH, 16)),
                p['bias'].reshape(1, HH))

    (lng1, lnb1, wlt1, bl1, wrt1, br1, wet1, attr1, attv1, bias1) = layer_mats(p1)
    (lng2, lnb2, wlt2, bl2, wrt2, br2, wet2, attr2, attv2, bias2) = layer_mats(p2)

    sumL, sumD = _pass0(dst, edge_attr)
    e1, e2 = _tc_edge_e(edge_attr, wet1, wet2)
    xl1, xr1, exs1, lm = _tc_prep(x, wpt, bp, lng1, lnb1, wlt1, bl1, wrt1,
                                  br1, wet1, attr1, sumL, sumD)
    msg1 = _pass1(src2, dst2, xl1, xr1, e1, attv1)
    acc1 = _pass2(dst2, msg1)
    xl2, xr2, exs2 = _tc_finish_prep(acc1, exs1, xl1, bias1, lng2,
                                     lnb2, wlt2, bl2, wrt2, br2, wet2,
                                     attr2, lm)
    msg2 = _pass1(src2, dst2, xl2, xr2, e2, attv2)
    acc2 = _pass2(dst2, msg2)
    cls, h = _tc_finish_cls(acc2, exs2, xl2, bias2, wct, bc)
    return (cls, h)


# speedup vs baseline: 1.5025x; 1.0359x over previous
import functools

import jax
import jax.numpy as jnp
from jax import lax
from jax.experimental import pallas as pl
from jax.experimental.pallas import tpu as pltpu
from jax.experimental.pallas import tpu_sc as plsc

NN = 50000
EE = 800000
DIN = 128
DE = 16
HH = 64
NEG = 0.2
NC = 2
NS = 16
NWK = NC * NS
EPT = EE // NWK

NCH1 = 195
EB1 = NCH1 * 128
XBASE = NWK * EB1
NROW = EE // 128

_MESH = plsc.VectorSubcoreMesh(core_axis_name="c", subcore_axis_name="s")

_F32 = jnp.float32
_I32 = jnp.int32

_CP = 3128
_CP_LAST = NN - (NS - 1) * _CP
_ZR = 200


def _fill_vec(ref, rows, cols, vec):
    nslot = cols // 16

    def body(i, _):
        r = i // nslot
        s = i % nslot
        ref[r, pl.ds(s * 16, 16)] = vec
        return 0

    lax.fori_loop(0, rows * nslot, body, 0)


def _zero_idx(ref, n):
    z = jnp.zeros((16,), _I32)
    for i in range(n // 16):
        ref[pl.ds(i * 16, 16)] = z


def _zero_spmem(acc, zb, sid):
    nchunk = NN // _ZR

    def body(j, _):
        c = sid + NS * j

        @pl.when(c < nchunk)
        def _():
            r0 = pl.multiple_of(c * _ZR, 8)
            pltpu.sync_copy(zb, acc.at[pl.ds(r0, _ZR)])
        return 0

    lax.fori_loop(0, (nchunk + NS - 1) // NS, body, 0)


def _copy_out_rows(acc, out_at, sid):
    @pl.when(sid < NS - 1)
    def _():
        r0 = pl.multiple_of(sid * _CP, 8)
        pltpu.sync_copy(acc.at[pl.ds(r0, _CP)], out_at.at[pl.ds(r0, _CP)])

    @pl.when(sid == NS - 1)
    def _():
        r0 = (NS - 1) * _CP
        pltpu.sync_copy(acc.at[pl.ds(r0, _CP_LAST)],
                        out_at.at[pl.ds(r0, _CP_LAST)])


@functools.partial(
    pl.kernel,
    out_type=(
        jax.ShapeDtypeStruct((NC, NN, DE), _F32),
        jax.ShapeDtypeStruct((NC, NN, DE), _F32),
    ),
    mesh=_MESH,
    compiler_params=pltpu.CompilerParams(needs_layout_passes=False,
                                         use_tc_tiling_on_sc=False),
    scratch_types=[
        pltpu.VMEM((128,), _I32),
        pltpu.VMEM((48,), _I32),
        pltpu.VMEM((128, DE), _F32),
        pltpu.VMEM((48, DE), _F32),
        pltpu.VMEM((_ZR, DE), _F32),
        pltpu.VMEM_SHARED((NN, DE), _F32),
    ],
)
def _pass0(dst_hbm, ea_hbm, outL, outD, idx, idxt, ea, eat, zb, acc):
    cid = lax.axis_index("c")
    sid = lax.axis_index("s")
    wid = sid * NC + cid
    base0 = wid * EPT
    zv = jnp.zeros((16,), _F32)
    ov = jnp.ones((16,), _F32)

    _fill_vec(zb, _ZR, DE, zv)
    _zero_idx(idxt, 48)
    pltpu.sync_copy(dst_hbm.at[pl.ds(base0 + 195 * 128, 40)],
                    idxt.at[pl.ds(0, 40)])

    _zero_spmem(acc, zb, sid)
    plsc.subcore_barrier()

    def chunk1(k, _):
        b = base0 + k * 128
        pltpu.sync_copy(dst_hbm.at[pl.ds(b, 128)], idx)
        pltpu.sync_copy(ea_hbm.at[pl.ds(b, 128)], ea)
        pltpu.sync_copy(ea, acc.at[idx], add=True)
        return 0
    lax.fori_loop(0, 195, chunk1, 0)

    _fill_vec(eat, 48, DE, zv)
    pltpu.sync_copy(ea_hbm.at[pl.ds(base0 + 195 * 128, 40)],
                    eat.at[pl.ds(0, 40)])
    pltpu.sync_copy(eat, acc.at[idxt], add=True)

    plsc.subcore_barrier()
    _copy_out_rows(acc, outL.at[cid], sid)
    plsc.subcore_barrier()

    _zero_spmem(acc, zb, sid)
    plsc.subcore_barrier()

    _fill_vec(ea, 128, DE, ov)

    def chunk2(k, _):
        b = base0 + k * 128
        pltpu.sync_copy(dst_hbm.at[pl.ds(b, 128)], idx)
        pltpu.sync_copy(ea, acc.at[idx], add=True)
        return 0
    lax.fori_loop(0, 195, chunk2, 0)

    _fill_vec(eat, 40, DE, ov)
    pltpu.sync_copy(eat, acc.at[idxt], add=True)

    plsc.subcore_barrier()
    _copy_out_rows(acc, outD.at[cid], sid)


@functools.partial(
    pl.kernel,
    out_type=jax.ShapeDtypeStruct((EE, 5, DE), _F32),
    mesh=_MESH,
    compiler_params=pltpu.CompilerParams(needs_layout_passes=False,
                                         use_tc_tiling_on_sc=False),
    scratch_types=[
        pltpu.VMEM((128,), _I32), pltpu.VMEM((128,), _I32),
        pltpu.VMEM((128,), _I32), pltpu.VMEM((128,), _I32),
        pltpu.VMEM((128,), _I32), pltpu.VMEM((128,), _I32),
        pltpu.VMEM((128,), _I32), pltpu.VMEM((128,), _I32),
        pltpu.VMEM((128, HH), _F32), pltpu.VMEM((128, HH), _F32),
        pltpu.VMEM((128, HH), _F32), pltpu.VMEM((128, HH), _F32),
        pltpu.VMEM((128, HH), _F32), pltpu.VMEM((128, HH), _F32),
        pltpu.VMEM((128, 5, DE), _F32), pltpu.VMEM((128, 5, DE), _F32),
        pltpu.VMEM((HH, 16), _F32),
        pltpu.SemaphoreType.DMA,
        pltpu.SemaphoreType.DMA,
        pltpu.SemaphoreType.DMA,
    ],
)
def _pass1(src2, dst2, xl_hbm, xr_hbm, e_hbm, att_hbm, msg_out,
           S0, S1, S2, S3, D0, D1, D2, D3,
           bL0, bL1, bR0, bR1, bE0, bE1, bM0, bM1,
           attb, semI, semG, semO):
    cid = lax.axis_index("c")
    sid = lax.axis_index("s")
    wid = sid * NC + cid
    row0 = wid * NCH1
    eb0 = wid * EB1
    S = [S0, S1, S2, S3]
    D = [D0, D1, D2, D3]
    bL = [bL0, bL1]
    bR = [bR0, bR1]
    bE = [bE0, bE1]
    bM = [bM0, bM1]

    pltpu.sync_copy(att_hbm, attb)
    iot = lax.iota(_I32, 16)
    rows_list = [iot + g * 16 for g in range(8)]

    def fire_idx(x, j):
        pltpu.async_copy(src2.at[row0 + x], S[j], semI)
        pltpu.async_copy(dst2.at[row0 + x], D[j], semI)

    def drain_idx():
        pltpu.make_async_copy(src2.at[0], S[0], semI).wait()
        pltpu.make_async_copy(dst2.at[0], D[0], semI).wait()

    def fire_gather(x, j, p):
        b = pl.multiple_of(eb0 + x * 128, 8)
        pltpu.async_copy(xl_hbm.at[S[j]], bL[p], semG)
        pltpu.async_copy(xr_hbm.at[D[j]], bR[p], semG)
        pltpu.async_copy(e_hbm.at[pl.ds(b, 128)], bE[p], semG)

    def drain_gather(p):
        pltpu.make_async_copy(xl_hbm.at[S[0]], bL[p], semG).wait()
        pltpu.make_async_copy(xr_hbm.at[D[0]], bR[p], semG).wait()
        pltpu.make_async_copy(e_hbm.at[pl.ds(0, 128)], bE[p], semG).wait()

    def fire_out(x, p):
        b = pl.multiple_of(eb0 + x * 128, 8)
        pltpu.async_copy(bM[p], msg_out.at[pl.ds(b, 128)], semO)

    def drain_out(p):
        pltpu.make_async_copy(bM[p], msg_out.at[pl.ds(0, 128)], semO).wait()

    def compute(p):
        bLp, bRp, bEp, bMp = bL[p], bR[p], bE[p], bM[p]

        def dbody(d, accs):
            dv = jnp.full((16,), d, _I32)
            ad = plsc.load_gather(attb, [dv, iot])
            new = []
            for g in range(8):
                rows = rows_list[g]
                xld = plsc.load_gather(bLp, [rows, dv])
                xrd = plsc.load_gather(bRp, [rows, dv])
                ed = plsc.load_gather(bEp, [rows, dv])
                z = xld + xrd + ed
                z = jnp.maximum(z, NEG * z)
                new.append(accs[g] + ad * z)
            return tuple(new)

        def dbody2(d2, accs):
            return dbody(d2 * 2 + 1, dbody(d2 * 2, accs))

        accs = lax.fori_loop(0, HH // 2, dbody2,
                             tuple(jnp.zeros((16,), _F32) for _ in range(8)))
        exs = [jnp.exp(a) for a in accs]

        def mbody(d, _):
            dv = jnp.full((16,), d, _I32)
            for q in range(4):
                qv = jnp.full((16,), q, _I32)
                cv = dv + q * DE
                for g in range(8):
                    rows = rows_list[g]
                    xld = plsc.load_gather(bLp, [rows, cv])
                    plsc.store_scatter(bMp, [rows, qv, dv], exs[g] * xld)
            return 0

        def mbody2(d2, _):
            mbody(d2 * 2, 0)
            mbody(d2 * 2 + 1, 0)
            return 0
        lax.fori_loop(0, DE // 2, mbody2, 0)

        qv4 = jnp.full((16,), 4, _I32)

        def xbody(d, _):
            dv = jnp.full((16,), d, _I32)
            for g in range(8):
                plsc.store_scatter(bMp, [rows_list[g], qv4, dv], exs[g])
            return 0
        lax.fori_loop(0, DE, xbody, 0)

    fire_idx(0, 0)
    fire_idx(1, 1)
    drain_idx()
    fire_gather(0, 0, 0)

    def body(k4, _):
        x0 = k4 * 4
        for j in range(4):
            x = x0 + j
            p = j % 2
            fire_idx(x + 2, (j + 2) % 4)
            drain_idx()
            fire_gather(x + 1, (j + 1) % 4, (p + 1) % 2)
            drain_gather(p)

            @pl.when(x >= 2)
            def _():
                drain_out(p)
            compute(p)
            fire_out(x, p)
        return 0
    lax.fori_loop(0, (NCH1 - 3) // 4, body, 0)

    fire_idx(194, 2)
    drain_idx()
    fire_gather(193, 1, 1)
    drain_gather(0)
    drain_out(0)
    compute(0)
    fire_out(192, 0)

    drain_idx()
    fire_gather(194, 2, 0)
    drain_gather(1)
    drain_out(1)
    compute(1)
    fire_out(193, 1)

    drain_gather(0)
    drain_out(0)
    compute(0)
    fire_out(194, 0)

    drain_out(1)
    drain_out(0)

    @pl.when(wid < 10)
    def _():
        pltpu.sync_copy(src2.at[NWK * NCH1 + wid], S[3])
        pltpu.sync_copy(dst2.at[NWK * NCH1 + wid], D[3])
        xb = pl.multiple_of(XBASE + wid * 128, 8)
        pltpu.async_copy(xl_hbm.at[S[3]], bL[1], semG)
        pltpu.async_copy(xr_hbm.at[D[3]], bR[1], semG)
        pltpu.async_copy(e_hbm.at[pl.ds(xb, 128)], bE[1], semG)
        drain_gather(1)
        compute(1)
        pltpu.sync_copy(bM[1], msg_out.at[pl.ds(xb, 128)])


@functools.partial(
    pl.kernel,
    out_type=jax.ShapeDtypeStruct((6, NN, DE), _F32),
    mesh=_MESH,
    compiler_params=pltpu.CompilerParams(needs_layout_passes=False,
                                         use_tc_tiling_on_sc=False),
    scratch_types=[
        pltpu.VMEM((128,), _I32), pltpu.VMEM((128,), _I32),
        pltpu.VMEM((128,), _I32), pltpu.VMEM((128,), _I32),
        pltpu.VMEM((128, DE), _F32), pltpu.VMEM((128, DE), _F32),
        pltpu.VMEM((128, DE), _F32), pltpu.VMEM((128, DE), _F32),
        pltpu.VMEM((_ZR, DE), _F32),
        pltpu.VMEM_SHARED((NN, DE), _F32),
        pltpu.SemaphoreType.DMA,
        pltpu.SemaphoreType.DMA,
        pltpu.SemaphoreType.DMA,
    ],
)
def _pass2(dst2, msg_hbm, acc_out,
           S0, S1, S2, S3, B0, B1, B2, B3, zb, accS, semI, semG, semS):
    cid = lax.axis_index("c")
    sid = lax.axis_index("s")
    S = [S0, S1, S2, S3]
    B = [B0, B1, B2, B3]
    _fill_vec(zb, _ZR, DE, jnp.zeros((16,), _F32))

    def run_phase(q, qout, row_base, eb_base, nch, n_extra, extra_row,
                  extra_eb):
        _zero_spmem(accS, zb, sid)
        plsc.subcore_barrier()

        def fire_idx(x, j):
            pltpu.async_copy(dst2.at[row_base + x], S[j], semI)

        def drain_idx():
            pltpu.make_async_copy(dst2.at[0], S[0], semI).wait()

        def fire_read(x, p):
            b = pl.multiple_of(eb_base + x * 128, 8)
            pltpu.async_copy(msg_hbm.at[pl.ds(b, 128), q], B[p], semG)

        def drain_read(p):
            pltpu.make_async_copy(msg_hbm.at[pl.ds(0, 128), q], B[p],
                                  semG).wait()

        def fire_scat(x, j, p):
            pltpu.async_copy(B[p], accS.at[S[j]], semS, add=True)

        def drain_scat():
            pltpu.make_async_copy(B[0], accS.at[S[0]], semS).wait()

        fire_idx(0, 0)
        fire_idx(1, 1)
        drain_idx()
        fire_read(0, 0)

        def body(k4, _):
            x0 = k4 * 4
            for j in range(4):
                x = x0 + j

                @pl.when(x >= 1)
                def _():
                    drain_scat()
                fire_idx(x + 2, (j + 2) % 4)
                drain_idx()
                fire_read(x + 1, (j + 1) % 4)
                drain_read(j)
                fire_scat(x, j, j)
            return 0
        nbody = (nch - 2) // 4
        lax.fori_loop(0, nbody, body, 0)

        for x in range(nbody * 4, nch):
            j = x % 4
            drain_scat()
            if x + 2 < nch:
                fire_idx(x + 2, (j + 2) % 4)
            if x + 1 < nch:
                drain_idx()
                fire_read(x + 1, (j + 1) % 4)
            drain_read(j)
            fire_scat(x, j, j)
        drain_scat()

        @pl.when(sid < n_extra)
        def _():
            pltpu.sync_copy(dst2.at[extra_row], S[0])
            xb = pl.multiple_of(extra_eb, 8)
            pltpu.sync_copy(msg_hbm.at[pl.ds(xb, 128), q], B[0])
            pltpu.sync_copy(B[0], accS.at[S[0]], add=True)

        plsc.subcore_barrier()
        _copy_out_rows(accS, acc_out.at[qout], sid)
        plsc.subcore_barrier()

    for ph in range(2):
        q = cid * 2 + ph
        run_phase(q, q,
                  sid * 390, sid * (390 * 128), 390,
                  10, NWK * NCH1 + sid, XBASE + sid * 128)

    run_phase(4, 4 + cid,
              cid * 3125 + sid * 195,
              cid * 400000 + sid * (195 * 128), 195,
              5, cid * 3125 + 3120 + sid,
              cid * 400000 + 399360 + sid * 128)


_RB = 1000
_REB = 2000


def _prep_body(x, wpt, bp, lng, lnb, wlt, bl, wrt, br, wet, attr, sl, sd,
               xl_o, xr_o, ex_o, lm_o):
    h = jnp.dot(x[...], wpt[...], preferred_element_type=_F32) + bp[...]
    mu = jnp.mean(h, axis=-1, keepdims=True)
    var = jnp.mean((h - mu) ** 2, axis=-1, keepdims=True)
    hn = (h - mu) * lax.rsqrt(var + 1e-5) * lng[...] + lnb[...]
    xl = jnp.dot(hn, wlt[...], preferred_element_type=_F32) + bl[...]
    xr = jnp.dot(hn, wrt[...], preferred_element_type=_F32) + br[...]
    deg = sd[...][0, :, 0:1] + sd[...][1, :, 0:1]
    lm = (sl[...][0] + sl[...][1]) / jnp.maximum(deg, 1.0)
    el = jnp.dot(lm, wet[...], preferred_element_type=_F32)
    z = xl + xr + el
    z = jnp.maximum(z, NEG * z)
    logit = jnp.sum(z * attr[...], axis=-1, keepdims=True)
    xl_o[...] = xl
    xr_o[...] = xr
    ex_o[...] = jnp.exp(logit)
    lm_o[...] = lm


def _finish_prep_body(acc, exs, xl, bias, lng, lnb, wlt, bl, wrt, br,
                      wet, attr, lm, xl_o, xr_o, ex_o):
    a = acc[...]
    num = jnp.concatenate([a[0], a[1], a[2], a[3]], axis=-1) + exs[...] * xl[...]
    d = a[4][:, 0:1] + a[5][:, 0:1] + exs[...] + 1e-16
    h = jnp.maximum(num / d + bias[...], 0.0)
    mu = jnp.mean(h, axis=-1, keepdims=True)
    var = jnp.mean((h - mu) ** 2, axis=-1, keepdims=True)
    hn = (h - mu) * lax.rsqrt(var + 1e-5) * lng[...] + lnb[...]
    xl2 = jnp.dot(hn, wlt[...], preferred_element_type=_F32) + bl[...]
    xr2 = jnp.dot(hn, wrt[...], preferred_element_type=_F32) + br[...]
    el = jnp.dot(lm[...], wet[...], preferred_element_type=_F32)
    z = xl2 + xr2 + el
    z = jnp.maximum(z, NEG * z)
    logit = jnp.sum(z * attr[...], axis=-1, keepdims=True)
    xl_o[...] = xl2
    xr_o[...] = xr2
    ex_o[...] = jnp.exp(logit)


def _finish_cls_body(acc, exs, xl, bias, wct, bc, cls_o, h_o):
    a = acc[...]
    num = jnp.concatenate([a[0], a[1], a[2], a[3]], axis=-1) + exs[...] * xl[...]
    d = a[4][:, 0:1] + a[5][:, 0:1] + exs[...] + 1e-16
    h = jnp.maximum(num / d + bias[...], 0.0)
    h_o[...] = h
    cls_o[...] = jnp.dot(h, wct[...], preferred_element_type=_F32) + bc[...]


def _edge_e_body(ea, w1, w2, e1_o, e2_o):
    a = ea[...]
    e1_o[...] = jnp.dot(a, w1[...], preferred_element_type=_F32)
    e2_o[...] = jnp.dot(a, w2[...], preferred_element_type=_F32)


def _row_spec(shape):
    nd = len(shape)
    if nd == 2:
        return pl.BlockSpec((_RB, shape[1]), lambda i: (i, 0))
    return pl.BlockSpec((shape[0], _RB, shape[2]), lambda i: (0, i, 0))


def _const_spec(shape):
    return pl.BlockSpec(shape, lambda i: tuple(0 for _ in shape))


def _tc_edge_e(ea, wet1, wet2):
    return pl.pallas_call(
        _edge_e_body,
        grid=(EE // _REB,),
        in_specs=[pl.BlockSpec((_REB, DE), lambda i: (i, 0)),
                  _const_spec((DE, HH)), _const_spec((DE, HH))],
        out_specs=[pl.BlockSpec((_REB, HH), lambda i: (i, 0)),
                   pl.BlockSpec((_REB, HH), lambda i: (i, 0))],
        out_shape=[jax.ShapeDtypeStruct((EE, HH), _F32),
                   jax.ShapeDtypeStruct((EE, HH), _F32)],
    )(ea, wet1, wet2)


def _tc_prep(x, wpt, bp, lng, lnb, wlt, bl, wrt, br, wet, attr, sumL, sumD):
    return pl.pallas_call(
        _prep_body,
        grid=(NN // _RB,),
        in_specs=[pl.BlockSpec((_RB, DIN), lambda i: (i, 0)),
                  _const_spec((DIN, HH)), _const_spec((1, HH)),
                  _const_spec((1, HH)), _const_spec((1, HH)),
                  _const_spec((HH, HH)), _const_spec((1, HH)),
                  _const_spec((HH, HH)), _const_spec((1, HH)),
                  _const_spec((DE, HH)), _const_spec((1, HH)),
                  _row_spec((NC, NN, DE)), _row_spec((NC, NN, DE))],
        out_specs=[pl.BlockSpec((_RB, HH), lambda i: (i, 0)),
                   pl.BlockSpec((_RB, HH), lambda i: (i, 0)),
                   pl.BlockSpec((_RB, 1), lambda i: (i, 0)),
                   pl.BlockSpec((_RB, DE), lambda i: (i, 0))],
        out_shape=[jax.ShapeDtypeStruct((NN, HH), _F32),
                   jax.ShapeDtypeStruct((NN, HH), _F32),
                   jax.ShapeDtypeStruct((NN, 1), _F32),
                   jax.ShapeDtypeStruct((NN, DE), _F32)],
    )(x, wpt, bp, lng, lnb, wlt, bl, wrt, br, wet, attr, sumL, sumD)


def _tc_finish_prep(acc, exs, xl, bias, lng, lnb, wlt, bl, wrt, br,
                    wet, attr, lm):
    return pl.pallas_call(
        _finish_prep_body,
        grid=(NN // _RB,),
        in_specs=[_row_spec((6, NN, DE)),
                  pl.BlockSpec((_RB, 1), lambda i: (i, 0)),
                  pl.BlockSpec((_RB, HH), lambda i: (i, 0)),
                  _const_spec((1, HH)),
                  _const_spec((1, HH)), _const_spec((1, HH)),
                  _const_spec((HH, HH)), _const_spec((1, HH)),
                  _const_spec((HH, HH)), _const_spec((1, HH)),
                  _const_spec((DE, HH)), _const_spec((1, HH)),
                  pl.BlockSpec((_RB, DE), lambda i: (i, 0))],
        out_specs=[pl.BlockSpec((_RB, HH), lambda i: (i, 0)),
                   pl.BlockSpec((_RB, HH), lambda i: (i, 0)),
                   pl.BlockSpec((_RB, 1), lambda i: (i, 0))],
        out_shape=[jax.ShapeDtypeStruct((NN, HH), _F32),
                   jax.ShapeDtypeStruct((NN, HH), _F32),
                   jax.ShapeDtypeStruct((NN, 1), _F32)],
    )(acc, exs, xl, bias, lng, lnb, wlt, bl, wrt, br, wet, attr, lm)


def _tc_finish_cls(acc, exs, xl, bias, wct, bc):
    return pl.pallas_call(
        _finish_cls_body,
        grid=(NN // _RB,),
        in_specs=[_row_spec((6, NN, DE)),
                  pl.BlockSpec((_RB, 1), lambda i: (i, 0)),
                  pl.BlockSpec((_RB, HH), lambda i: (i, 0)),
                  _const_spec((1, HH)),
                  _const_spec((HH, 64)), _const_spec((1, 64))],
        out_specs=[pl.BlockSpec((_RB, 64), lambda i: (i, 0)),
                   pl.BlockSpec((_RB, HH), lambda i: (i, 0))],
        out_shape=[jax.ShapeDtypeStruct((NN, 64), _F32),
                   jax.ShapeDtypeStruct((NN, HH), _F32)],
    )(acc, exs, xl, bias, wct, bc)


def kernel(x, edge_index, edge_attr, params):
    src = edge_index[0]
    dst = edge_index[1]
    src2 = src.reshape(NROW, 128)
    dst2 = dst.reshape(NROW, 128)
    p1, p2 = params['layers']

    wpt = params['Wp'].T
    bp = params['bp'].reshape(1, HH)
    wct = params['Wc'].T
    bc = params['bc'].reshape(1, 64)

    def layer_mats(p):
        return (p['ln_g'].reshape(1, HH), p['ln_b'].reshape(1, HH),
                p['Wl'].T, p['bl'].reshape(1, HH),
                p['Wr'].T, p['br'].reshape(1, HH),
                p['We'].T, p['att'].reshape(1, HH),
                jnp.broadcast_to(p['att'].reshape(HH, 1), (HH, 16)),
                p['bias'].reshape(1, HH))

    (lng1, lnb1, wlt1, bl1, wrt1, br1, wet1, attr1, attv1, bias1) = layer_mats(p1)
    (lng2, lnb2, wlt2, bl2, wrt2, br2, wet2, attr2, attv2, bias2) = layer_mats(p2)

    sumL, sumD = _pass0(dst, edge_attr)
    e1, e2 = _tc_edge_e(edge_attr, wet1, wet2)
    xl1, xr1, exs1, lm = _tc_prep(x, wpt, bp, lng1, lnb1, wlt1, bl1, wrt1,
                                  br1, wet1, attr1, sumL, sumD)
    msg1 = _pass1(src2, dst2, xl1, xr1, e1, attv1)
    acc1 = _pass2(dst2, msg1)
    xl2, xr2, exs2 = _tc_finish_prep(acc1, exs1, xl1, bias1, lng2,
                                     lnb2, wlt2, bl2, wrt2, br2, wet2,
                                     attr2, lm)
    msg2 = _pass1(src2, dst2, xl2, xr2, e2, attv2)
    acc2 = _pass2(dst2, msg2)
    cls, h = _tc_finish_cls(acc2, exs2, xl2, bias2, wct, bc)
    return (cls, h)
